# baseline (speedup 1.0000x reference)
import functools

import jax
import jax.numpy as jnp
from jax import lax
from jax.experimental import pallas as pl
from jax.experimental.pallas import tpu as pltpu
from jax.experimental.pallas import tpu_sc as plsc

T = 2048
C = 768
E = 8
H = 4 * C
TM = 128
NT = T // TM + 8
TPAD = NT * TM
TR = 256


_NC, _NS = 2, 16
_NW = _NC * _NS


def _sc_mesh():
    return plsc.VectorSubcoreMesh(core_axis_name="c", subcore_axis_name="s",
                                  num_cores=_NC, num_subcores=_NS)


def _row_gather1(src, idx, n_out):
    bpw = n_out // _NW
    f32 = jnp.float32

    @functools.partial(
        pl.kernel, out_type=jax.ShapeDtypeStruct((n_out, C), f32),
        mesh=_sc_mesh(),
        scratch_types=[pltpu.VMEM((bpw,), jnp.int32),
                       pltpu.VMEM((bpw, C), f32),
                       pltpu.SemaphoreType.DMA])
    def k(src_hbm, idx_hbm, out_hbm, idx_v, rows_v, sem):
        wid = lax.axis_index("s") * _NC + lax.axis_index("c")
        base = wid * bpw
        pltpu.sync_copy(idx_hbm.at[pl.ds(base, bpw)], idx_v)
        pltpu.async_copy(src_hbm.at[idx_v], rows_v, sem).wait()
        pltpu.sync_copy(rows_v, out_hbm.at[pl.ds(base, bpw)])

    return k(src, idx)


def _ln(z, g, b):
    m = jnp.mean(z, axis=-1, keepdims=True)
    v = jnp.mean((z - m) ** 2, axis=-1, keepdims=True)
    return (z - m) * lax.rsqrt(v + 1e-5) * g + b


def _prologue_body(x_ref, xp_ref, wrkvs_ref, wo_ref, wroute_ref, confb_ref,
                   cap_ref, ln1g_ref, ln1b_ref, ln2g_ref, ln2b_ref,
                   x1_ref, h_ref, st_ref, v_ref, win_ref, cost_ref, diff_ref,
                   aff_ref, scale_ref):
    i = pl.program_id(0)
    g1, b1 = ln1g_ref[...], ln1b_ref[...]
    h1 = _ln(x_ref[...], g1, b1)
    h1s = _ln(xp_ref[...], g1, b1)
    row = lax.broadcasted_iota(jnp.int32, h1s.shape, 0) + i * TR
    h1s = jnp.where(row == 0, 0.0, h1s)
    mix = 0.5 * (h1 + h1s)
    rkvs = jnp.dot(mix, wrkvs_ref[...], preferred_element_type=jnp.float32)
    r = jax.nn.sigmoid(rkvs[:, 0:C])
    k = rkvs[:, C:2 * C]
    v = rkvs[:, 2 * C:3 * C]
    st = rkvs[:, 3 * C:4 * C]
    att = jnp.dot(r * k * v, wo_ref[...], preferred_element_type=jnp.float32)
    x1 = x_ref[...] + att
    h = _ln(x1, ln2g_ref[...], ln2b_ref[...])
    route = jnp.dot(h, wroute_ref[...], preferred_element_type=jnp.float32)
    conf = jax.nn.sigmoid(route[:, 0:E] + confb_ref[...])
    diff = jax.nn.sigmoid(route[:, E:E + 1])
    aff = route[:, E + 1:E + 1 + E]
    bids = conf * cap_ref[...] + 0.01 * aff
    maxb = jnp.max(bids, axis=-1, keepdims=True)
    eio = lax.broadcasted_iota(jnp.int32, bids.shape, 1)
    win = jnp.min(jnp.where(bids >= maxb, eio, E), axis=-1, keepdims=True)
    wb = jnp.sum(jnp.where(eio == win, conf, 0.0), axis=-1, keepdims=True)
    x1_ref[...] = x1
    h_ref[...] = h
    st_ref[...] = st
    v_ref[...] = v
    win_ref[...] = win
    cost_ref[...] = maxb * diff
    diff_ref[...] = diff
    aff_ref[...] = aff
    scale_ref[...] = wb / (wb + 1e-6)


def _prologue(x2d, xp2d, wrkvs, wo, wroute, confb, cap, g1, b1, g2, b2):
    rows = lambda i: (i, 0)
    whole = lambda i: (0, 0)
    f32 = jnp.float32
    return pl.pallas_call(
        _prologue_body,
        grid=(T // TR,),
        in_specs=[
            pl.BlockSpec((TR, C), rows),
            pl.BlockSpec((TR, C), rows),
            pl.BlockSpec((C, 4 * C), whole),
            pl.BlockSpec((C, C), whole),
            pl.BlockSpec((C, 2 * E + 1), whole),
            pl.BlockSpec((1, E), whole),
            pl.BlockSpec((1, E), whole),
            pl.BlockSpec((1, C), whole),
            pl.BlockSpec((1, C), whole),
            pl.BlockSpec((1, C), whole),
            pl.BlockSpec((1, C), whole),
        ],
        out_specs=[
            pl.BlockSpec((TR, C), rows),
            pl.BlockSpec((TR, C), rows),
            pl.BlockSpec((TR, C), rows),
            pl.BlockSpec((TR, C), rows),
            pl.BlockSpec((TR, 1), rows),
            pl.BlockSpec((TR, 1), rows),
            pl.BlockSpec((TR, 1), rows),
            pl.BlockSpec((TR, E), rows),
            pl.BlockSpec((TR, 1), rows),
        ],
        out_shape=[
            jax.ShapeDtypeStruct((T, C), f32),
            jax.ShapeDtypeStruct((T, C), f32),
            jax.ShapeDtypeStruct((T, C), f32),
            jax.ShapeDtypeStruct((T, C), f32),
            jax.ShapeDtypeStruct((T, 1), jnp.int32),
            jax.ShapeDtypeStruct((T, 1), f32),
            jax.ShapeDtypeStruct((T, 1), f32),
            jax.ShapeDtypeStruct((T, E), f32),
            jax.ShapeDtypeStruct((T, 1), f32),
        ],
    )(x2d, xp2d, wrkvs, wo, wroute, confb, cap, g1, b1, g2, b2)


def _ffn_body(te_ref, h_ref, st_ref, x1_ref, sc_ref, vm_ref,
              w1_ref, b1_ref, w2_ref, b2_ref, ws1_ref, wrec_ref,
              out_ref, rec_ref):
    i = pl.program_id(0)
    e = te_ref[i]
    h = h_ref[...]
    hb = h.astype(jnp.bfloat16)
    base = jnp.dot(hb, w1_ref[0], preferred_element_type=jnp.float32) + b1_ref[0]

    @pl.when(i == 0)
    def _init():
        rec_ref[...] = jnp.zeros_like(rec_ref)

    @pl.when(e == E - 1)
    def _last_expert():
        st = st_ref[...]
        hid = jax.nn.relu(
            base + jnp.dot(st.astype(jnp.bfloat16), ws1_ref[...],
                           preferred_element_type=jnp.float32))
        out = jnp.dot(hid.astype(jnp.bfloat16), w2_ref[0],
                      preferred_element_type=jnp.float32) + b2_ref[0]
        out_ref[...] = x1_ref[...] + out * sc_ref[...]
        r = jnp.dot(h, wrec_ref[...], preferred_element_type=jnp.float32) - st
        rec_ref[...] += jnp.sum(
            jnp.sum(r * r, axis=-1, keepdims=True) * vm_ref[...]).reshape(1, 1)

    @pl.when(e != E - 1)
    def _ffn_expert():
        hr = jax.nn.relu(base)
        out = jnp.dot((hr * hr).astype(jnp.bfloat16), w2_ref[0],
                      preferred_element_type=jnp.float32) + b2_ref[0]
        out_ref[...] = x1_ref[...] + out * sc_ref[...]


def _ffn(tile_expert, h_s, st_s, x1_s, sc_s, vm_s, w1, b1e, w2, b2e, ws1, wrec):
    rows = lambda i, te: (i, 0)
    byexp3 = lambda i, te: (te[i], 0, 0)
    whole = lambda i, te: (0, 0)
    f32 = jnp.float32
    grid_spec = pltpu.PrefetchScalarGridSpec(
        num_scalar_prefetch=1,
        grid=(NT,),
        in_specs=[
            pl.BlockSpec((TM, C), rows),
            pl.BlockSpec((TM, C), rows),
            pl.BlockSpec((TM, C), rows),
            pl.BlockSpec((TM, 1), rows),
            pl.BlockSpec((TM, 1), rows),
            pl.BlockSpec((1, C, H), byexp3),
            pl.BlockSpec((1, 1, H), byexp3),
            pl.BlockSpec((1, H, C), byexp3),
            pl.BlockSpec((1, 1, C), byexp3),
            pl.BlockSpec((C, H), whole),
            pl.BlockSpec((C, C), whole),
        ],
        out_specs=[
            pl.BlockSpec((TM, C), rows),
            pl.BlockSpec((1, 1), whole),
        ],
    )
    return pl.pallas_call(
        _ffn_body,
        grid_spec=grid_spec,
        out_shape=[
            jax.ShapeDtypeStruct((TPAD, C), f32),
            jax.ShapeDtypeStruct((1, 1), f32),
        ],
    )(tile_expert, h_s, st_s, x1_s, sc_s, vm_s, w1, b1e, w2, b2e, ws1, wrec)


def kernel(x, v_first, capital_shares, ln1_g, ln1_b, ln2_g, ln2_b, Wr, Wk, Wv,
           Wo, Ws, W1, b1, W2, b2, Ws1, Wrec, conf_w, conf_b, Wd, Wa):
    f32 = jnp.float32
    x2d = x.reshape(T, C)
    xp2d = jnp.concatenate([jnp.zeros((1, C), f32), x2d[:-1]], axis=0)
    wrkvs = jnp.concatenate([Wr, Wk, Wv, Ws], axis=1)
    wroute = jnp.concatenate([conf_w.T, Wd, Wa], axis=1)

    (x1, h, st, v, win2, cost2, diff2, aff, scale2) = _prologue(
        x2d, xp2d, wrkvs, Wo, wroute, conf_b.reshape(1, E),
        capital_shares.reshape(1, E), ln1_g.reshape(1, C), ln1_b.reshape(1, C),
        ln2_g.reshape(1, C), ln2_b.reshape(1, C))

    winners = win2[:, 0]
    counts = jnp.sum((winners[:, None] == jnp.arange(E)[None, :]).astype(jnp.int32), axis=0)
    tiles_e = (counts + TM - 1) // TM
    cum_tiles = jnp.cumsum(tiles_e)
    pstart = (cum_tiles - tiles_e) * TM
    offs = jnp.cumsum(counts) - counts
    ti = jnp.arange(NT)
    tile_expert = jnp.minimum(
        jnp.sum((ti[:, None] >= cum_tiles[None, :]).astype(jnp.int32), axis=1),
        E - 1).astype(jnp.int32)
    sort_idx = jnp.argsort(winners)
    rank = jnp.argsort(sort_idx)
    inv_perm = (pstart[winners] + (rank - offs[winners])).astype(jnp.int32)
    qi = jnp.arange(TPAD)
    qe = tile_expert[qi // TM]
    j = qi - pstart[qe]
    valid = j < counts[qe]
    src_row = jnp.where(
        valid, sort_idx[jnp.clip(offs[qe] + j, 0, T - 1)], 0).astype(jnp.int32)
    vm_s = valid.astype(f32)[:, None]
    sc_s = scale2[src_row]

    h_s = _row_gather1(h, src_row, TPAD)
    st_s = _row_gather1(st, src_row, TPAD)
    x1_s = _row_gather1(x1, src_row, TPAD)

    ffn_out, rec_sum = _ffn(tile_expert, h_s, st_s, x1_s, sc_s, vm_s,
                            W1.astype(jnp.bfloat16), b1.reshape(E, 1, H),
                            W2.astype(jnp.bfloat16), b2.reshape(E, 1, C),
                            Ws1.astype(jnp.bfloat16), Wrec)

    x_out = _row_gather1(ffn_out, inv_perm, T)

    cnt7 = counts[E - 1]
    recon = jnp.where(cnt7 > 0, rec_sum[0, 0] / (cnt7 * C).astype(f32), 0.0)

    return (x_out.reshape(1, T, C), v.reshape(1, T, C), winners.reshape(1, T),
            cost2[:, 0].reshape(1, T), diff2.reshape(1, T, 1),
            aff.reshape(1, T, E), recon)


# speedup vs baseline: 1.6674x; 1.6674x over previous
import functools

import jax
import jax.numpy as jnp
from jax import lax
from jax.experimental import pallas as pl
from jax.experimental.pallas import tpu as pltpu
from jax.experimental.pallas import tpu_sc as plsc

T = 2048
C = 768
E = 8
H = 4 * C
TM = 128
NT = T // TM + 8
TPAD = NT * TM
TR = 256


_NC, _NS = 2, 16
_NW = _NC * _NS


def _sc_mesh():
    return plsc.VectorSubcoreMesh(core_axis_name="c", subcore_axis_name="s",
                                  num_cores=_NC, num_subcores=_NS)


def _row_gather1(src, idx, n_out):
    bpw = n_out // _NW
    f32 = jnp.float32

    @functools.partial(
        pl.kernel, out_type=jax.ShapeDtypeStruct((n_out, C), f32),
        mesh=_sc_mesh(),
        scratch_types=[pltpu.VMEM((bpw,), jnp.int32),
                       pltpu.VMEM((bpw, C), f32),
                       pltpu.SemaphoreType.DMA])
    def k(src_hbm, idx_hbm, out_hbm, idx_v, rows_v, sem):
        wid = lax.axis_index("s") * _NC + lax.axis_index("c")
        base = wid * bpw
        pltpu.sync_copy(idx_hbm.at[pl.ds(base, bpw)], idx_v)
        pltpu.async_copy(src_hbm.at[idx_v], rows_v, sem).wait()
        pltpu.sync_copy(rows_v, out_hbm.at[pl.ds(base, bpw)])

    return k(src, idx)


def _ln(z, g, b):
    m = jnp.mean(z, axis=-1, keepdims=True)
    v = jnp.mean((z - m) ** 2, axis=-1, keepdims=True)
    return (z - m) * lax.rsqrt(v + 1e-5) * g + b


def _prologue_body(x_ref, xp_ref, wrkvs_ref, wo_ref, wroute_ref, confb_ref,
                   cap_ref, ln1g_ref, ln1b_ref, ln2g_ref, ln2b_ref, wrec_ref,
                   x1_ref, h_ref, st_ref, v_ref, win_ref, cost_ref, diff_ref,
                   aff_ref, scale_ref, rec_ref):
    i = pl.program_id(0)
    g1, b1 = ln1g_ref[...], ln1b_ref[...]
    h1 = _ln(x_ref[...], g1, b1)
    h1s = _ln(xp_ref[...], g1, b1)
    row = lax.broadcasted_iota(jnp.int32, h1s.shape, 0) + i * TR
    h1s = jnp.where(row == 0, 0.0, h1s)
    mix = 0.5 * (h1 + h1s)
    rkvs = jnp.dot(mix, wrkvs_ref[...], preferred_element_type=jnp.float32)
    r = jax.nn.sigmoid(rkvs[:, 0:C])
    k = rkvs[:, C:2 * C]
    v = rkvs[:, 2 * C:3 * C]
    st = rkvs[:, 3 * C:4 * C]
    att = jnp.dot(r * k * v, wo_ref[...], preferred_element_type=jnp.float32)
    x1 = x_ref[...] + att
    h = _ln(x1, ln2g_ref[...], ln2b_ref[...])
    route = jnp.dot(h, wroute_ref[...], preferred_element_type=jnp.float32)
    conf = jax.nn.sigmoid(route[:, 0:E] + confb_ref[...])
    diff = jax.nn.sigmoid(route[:, E:E + 1])
    aff = route[:, E + 1:E + 1 + E]
    bids = conf * cap_ref[...] + 0.01 * aff
    maxb = jnp.max(bids, axis=-1, keepdims=True)
    eio = lax.broadcasted_iota(jnp.int32, bids.shape, 1)
    win = jnp.min(jnp.where(bids >= maxb, eio, E), axis=-1, keepdims=True)
    wb = jnp.sum(jnp.where(eio == win, conf, 0.0), axis=-1, keepdims=True)
    x1_ref[...] = x1
    h_ref[...] = h
    st_ref[...] = st
    v_ref[...] = v
    win_ref[...] = win
    cost_ref[...] = maxb * diff
    diff_ref[...] = diff
    aff_ref[...] = aff
    scale_ref[...] = wb / (wb + 1e-6)

    @pl.when(i == 0)
    def _init():
        rec_ref[...] = jnp.zeros_like(rec_ref)

    rr = jnp.dot(h, wrec_ref[...], preferred_element_type=jnp.float32) - st
    m7 = (win == E - 1).astype(jnp.float32)
    rec_ref[...] += jnp.sum(
        jnp.sum(rr * rr, axis=-1, keepdims=True) * m7).reshape(1, 1)


def _prologue(x2d, xp2d, wrkvs, wo, wroute, confb, cap, g1, b1, g2, b2, wrec):
    rows = lambda i: (i, 0)
    whole = lambda i: (0, 0)
    f32 = jnp.float32
    return pl.pallas_call(
        _prologue_body,
        grid=(T // TR,),
        in_specs=[
            pl.BlockSpec((TR, C), rows),
            pl.BlockSpec((TR, C), rows),
            pl.BlockSpec((C, 4 * C), whole),
            pl.BlockSpec((C, C), whole),
            pl.BlockSpec((C, 2 * E + 1), whole),
            pl.BlockSpec((1, E), whole),
            pl.BlockSpec((1, E), whole),
            pl.BlockSpec((1, C), whole),
            pl.BlockSpec((1, C), whole),
            pl.BlockSpec((1, C), whole),
            pl.BlockSpec((1, C), whole),
            pl.BlockSpec((C, C), whole),
        ],
        out_specs=[
            pl.BlockSpec((TR, C), rows),
            pl.BlockSpec((TR, C), rows),
            pl.BlockSpec((TR, C), rows),
            pl.BlockSpec((TR, C), rows),
            pl.BlockSpec((TR, 1), rows),
            pl.BlockSpec((TR, 1), rows),
            pl.BlockSpec((TR, 1), rows),
            pl.BlockSpec((TR, E), rows),
            pl.BlockSpec((TR, 1), rows),
            pl.BlockSpec((1, 1), whole),
        ],
        out_shape=[
            jax.ShapeDtypeStruct((T, C), f32),
            jax.ShapeDtypeStruct((T, C), f32),
            jax.ShapeDtypeStruct((T, C), f32),
            jax.ShapeDtypeStruct((T, C), f32),
            jax.ShapeDtypeStruct((T, 1), jnp.int32),
            jax.ShapeDtypeStruct((T, 1), f32),
            jax.ShapeDtypeStruct((T, 1), f32),
            jax.ShapeDtypeStruct((T, E), f32),
            jax.ShapeDtypeStruct((T, 1), f32),
            jax.ShapeDtypeStruct((1, 1), f32),
        ],
    )(x2d, xp2d, wrkvs, wo, wroute, confb, cap, g1, b1, g2, b2, wrec)


def _ffn_body(te_ref, h_ref, st_ref, x1_ref, sc_ref,
              w1_ref, b1_ref, w2_ref, b2_ref, ws1_ref,
              out_ref):
    i = pl.program_id(0)
    e = te_ref[i]
    h = h_ref[...]
    stz = st_ref[...]
    base = (jnp.dot(h, w1_ref[0], preferred_element_type=jnp.float32)
            + jnp.dot(stz, ws1_ref[...], preferred_element_type=jnp.float32)
            + b1_ref[0])
    hr = jax.nn.relu(base)
    hid = jnp.where(e == E - 1, hr, hr * hr)
    out = jnp.dot(hid, w2_ref[0], preferred_element_type=jnp.float32) + b2_ref[0]
    out_ref[...] = x1_ref[...] + out * sc_ref[...]


def _ffn(tile_expert, h_s, st_s, x1_s, sc_s, w1, b1e, w2, b2e, ws1):
    rows = lambda i, te: (i, 0)
    byexp3 = lambda i, te: (te[i], 0, 0)
    whole = lambda i, te: (0, 0)
    f32 = jnp.float32
    grid_spec = pltpu.PrefetchScalarGridSpec(
        num_scalar_prefetch=1,
        grid=(NT,),
        in_specs=[
            pl.BlockSpec((TM, C), rows),
            pl.BlockSpec((TM, C), rows),
            pl.BlockSpec((TM, C), rows),
            pl.BlockSpec((TM, 1), rows),
            pl.BlockSpec((1, C, H), byexp3),
            pl.BlockSpec((1, 1, H), byexp3),
            pl.BlockSpec((1, H, C), byexp3),
            pl.BlockSpec((1, 1, C), byexp3),
            pl.BlockSpec((C, H), whole),
        ],
        out_specs=pl.BlockSpec((TM, C), rows),
        )
    return pl.pallas_call(
        _ffn_body,
        grid_spec=grid_spec,
        out_shape=jax.ShapeDtypeStruct((TPAD, C), f32),
    )(tile_expert, h_s, st_s, x1_s, sc_s, w1, b1e, w2, b2e, ws1)


def kernel(x, v_first, capital_shares, ln1_g, ln1_b, ln2_g, ln2_b, Wr, Wk, Wv,
           Wo, Ws, W1, b1, W2, b2, Ws1, Wrec, conf_w, conf_b, Wd, Wa):
    f32 = jnp.float32
    x2d = x.reshape(T, C)
    xp2d = jnp.concatenate([jnp.zeros((1, C), f32), x2d[:-1]], axis=0)
    wrkvs = jnp.concatenate([Wr, Wk, Wv, Ws], axis=1)
    wroute = jnp.concatenate([conf_w.T, Wd, Wa], axis=1)

    (x1, h, st, v, win2, cost2, diff2, aff, scale2, rec_sum) = _prologue(
        x2d, xp2d, wrkvs, Wo, wroute, conf_b.reshape(1, E),
        capital_shares.reshape(1, E), ln1_g.reshape(1, C), ln1_b.reshape(1, C),
        ln2_g.reshape(1, C), ln2_b.reshape(1, C), Wrec)

    winners = win2[:, 0]
    counts = jnp.sum((winners[:, None] == jnp.arange(E)[None, :]).astype(jnp.int32), axis=0)
    tiles_e = (counts + TM - 1) // TM
    cum_tiles = jnp.cumsum(tiles_e)
    pstart = (cum_tiles - tiles_e) * TM
    offs = jnp.cumsum(counts) - counts
    ti = jnp.arange(NT)
    tile_expert = jnp.minimum(
        jnp.sum((ti[:, None] >= cum_tiles[None, :]).astype(jnp.int32), axis=1),
        E - 1).astype(jnp.int32)
    sort_idx = jnp.argsort(winners)
    rank = jnp.argsort(sort_idx)
    inv_perm = (pstart[winners] + (rank - offs[winners])).astype(jnp.int32)
    qi = jnp.arange(TPAD)
    qe = tile_expert[qi // TM]
    j = qi - pstart[qe]
    valid = j < counts[qe]
    src_row = jnp.where(
        valid, sort_idx[jnp.clip(offs[qe] + j, 0, T - 1)],
        qi % T).astype(jnp.int32)
    sc_s = scale2[src_row]

    st_big = jnp.concatenate([st, jnp.zeros_like(st)], axis=0)
    src_row_st = jnp.where(qe == E - 1, src_row, T + (qi % T)).astype(jnp.int32)
    h_s = _row_gather1(h, src_row, TPAD)
    st_s = _row_gather1(st_big, src_row_st, TPAD)
    x1_s = _row_gather1(x1, src_row, TPAD)

    ffn_out = _ffn(tile_expert, h_s, st_s, x1_s, sc_s,
                   W1, b1.reshape(E, 1, H), W2, b2.reshape(E, 1, C), Ws1)

    x_out = _row_gather1(ffn_out, inv_perm, T)

    cnt7 = counts[E - 1]
    recon = jnp.where(cnt7 > 0, rec_sum[0, 0] / (cnt7 * C).astype(f32), 0.0)

    return (x_out.reshape(1, T, C), v.reshape(1, T, C), winners.reshape(1, T),
            cost2[:, 0].reshape(1, T), diff2.reshape(1, T, 1),
            aff.reshape(1, T, E), recon)


# speedup vs baseline: 1.9085x; 1.1446x over previous
import functools

import jax
import jax.numpy as jnp
from jax import lax
from jax.experimental import pallas as pl
from jax.experimental.pallas import tpu as pltpu
from jax.experimental.pallas import tpu_sc as plsc

T = 2048
C = 768
E = 8
H = 4 * C
TM = 128
NT = T // TM + 8
TPAD = NT * TM
TR = 256


_NC, _NS = 2, 16
_NW = _NC * _NS


def _sc_mesh():
    return plsc.VectorSubcoreMesh(core_axis_name="c", subcore_axis_name="s",
                                  num_cores=_NC, num_subcores=_NS)


def _row_gather1(src, idx, n_out):
    bpw = n_out // _NW
    f32 = jnp.float32

    @functools.partial(
        pl.kernel, out_type=jax.ShapeDtypeStruct((n_out, C), f32),
        mesh=_sc_mesh(),
        scratch_types=[pltpu.VMEM((bpw,), jnp.int32),
                       pltpu.VMEM((bpw, C), f32),
                       pltpu.SemaphoreType.DMA])
    def k(src_hbm, idx_hbm, out_hbm, idx_v, rows_v, sem):
        wid = lax.axis_index("s") * _NC + lax.axis_index("c")
        base = wid * bpw
        pltpu.sync_copy(idx_hbm.at[pl.ds(base, bpw)], idx_v)
        pltpu.async_copy(src_hbm.at[idx_v], rows_v, sem).wait()
        pltpu.sync_copy(rows_v, out_hbm.at[pl.ds(base, bpw)])

    return k(src, idx)


def _ln(z, g, b):
    m = jnp.mean(z, axis=-1, keepdims=True)
    v = jnp.mean((z - m) ** 2, axis=-1, keepdims=True)
    return (z - m) * lax.rsqrt(v + 1e-5) * g + b


def _prologue_body(x_ref, xp_ref, wrkvs_ref, wo_ref, wroute_ref, confb_ref,
                   cap_ref, ln1g_ref, ln1b_ref, ln2g_ref, ln2b_ref, wrec_ref,
                   x1_ref, h_ref, st_ref, v_ref, win_ref, cost_ref, diff_ref,
                   aff_ref, scale_ref, rec_ref, rank_ref, cnt_ref):
    i = pl.program_id(0)
    g1, b1 = ln1g_ref[...], ln1b_ref[...]
    h1 = _ln(x_ref[...], g1, b1)
    h1p = _ln(xp_ref[TR - 1:TR, :], g1, b1)
    h1s = jnp.concatenate([h1p, h1[:TR - 1, :]], axis=0)
    row = lax.broadcasted_iota(jnp.int32, h1s.shape, 0) + i * TR
    h1s = jnp.where(row == 0, 0.0, h1s)
    mix = 0.5 * (h1 + h1s)
    rkvs = jnp.dot(mix, wrkvs_ref[...], preferred_element_type=jnp.float32)
    r = jax.nn.sigmoid(rkvs[:, 0:C])
    k = rkvs[:, C:2 * C]
    v = rkvs[:, 2 * C:3 * C]
    st = rkvs[:, 3 * C:4 * C]
    att = jnp.dot(r * k * v, wo_ref[...], preferred_element_type=jnp.float32)
    x1 = x_ref[...] + att
    h = _ln(x1, ln2g_ref[...], ln2b_ref[...])
    route = jnp.dot(h, wroute_ref[...], preferred_element_type=jnp.float32)
    conf = jax.nn.sigmoid(route[:, 0:E] + confb_ref[...])
    diff = jax.nn.sigmoid(route[:, E:E + 1])
    aff = route[:, E + 1:E + 1 + E]
    bids = conf * cap_ref[...] + 0.01 * aff
    maxb = jnp.max(bids, axis=-1, keepdims=True)
    eio = lax.broadcasted_iota(jnp.int32, bids.shape, 1)
    win = jnp.min(jnp.where(bids >= maxb, eio, E), axis=-1, keepdims=True)
    wb = jnp.sum(jnp.where(eio == win, conf, 0.0), axis=-1, keepdims=True)
    x1_ref[...] = x1
    h_ref[...] = h
    st_ref[...] = st
    v_ref[...] = v
    win_ref[...] = win
    cost_ref[...] = maxb * diff
    diff_ref[...] = diff
    aff_ref[...] = aff
    scale_ref[...] = wb / (wb + 1e-6)

    @pl.when(i == 0)
    def _init():
        rec_ref[...] = jnp.zeros_like(rec_ref)
        cnt_ref[...] = jnp.zeros_like(cnt_ref)

    rr = jnp.dot(h, wrec_ref[...], preferred_element_type=jnp.float32) - st
    m7 = (win == E - 1).astype(jnp.float32)
    rec_ref[...] += jnp.sum(
        jnp.sum(rr * rr, axis=-1, keepdims=True) * m7).reshape(1, 1)

    oh = (eio == win).astype(jnp.float32)
    rio = lax.broadcasted_iota(jnp.int32, (TR, TR), 0)
    cio = lax.broadcasted_iota(jnp.int32, (TR, TR), 1)
    tri = (rio > cio).astype(jnp.float32)
    excl = jnp.dot(tri, oh, preferred_element_type=jnp.float32)
    base = cnt_ref[...].astype(jnp.float32)
    rank_ref[...] = jnp.sum(oh * (excl + base), axis=1,
                            keepdims=True).astype(jnp.int32)
    cnt_ref[...] += jnp.sum(oh, axis=0, keepdims=True).astype(jnp.int32)


def _prologue(x2d, xp2d, wrkvs, wo, wroute, confb, cap, g1, b1, g2, b2, wrec):
    rows = lambda i: (i, 0)
    prev = lambda i: (jnp.maximum(i - 1, 0), 0)
    whole = lambda i: (0, 0)
    f32 = jnp.float32
    return pl.pallas_call(
        _prologue_body,
        grid=(T // TR,),
        in_specs=[
            pl.BlockSpec((TR, C), rows),
            pl.BlockSpec((TR, C), prev),
            pl.BlockSpec((C, 4 * C), whole),
            pl.BlockSpec((C, C), whole),
            pl.BlockSpec((C, 2 * E + 1), whole),
            pl.BlockSpec((1, E), whole),
            pl.BlockSpec((1, E), whole),
            pl.BlockSpec((1, C), whole),
            pl.BlockSpec((1, C), whole),
            pl.BlockSpec((1, C), whole),
            pl.BlockSpec((1, C), whole),
            pl.BlockSpec((C, C), whole),
        ],
        out_specs=[
            pl.BlockSpec((TR, C), rows),
            pl.BlockSpec((TR, C), rows),
            pl.BlockSpec((TR, C), rows),
            pl.BlockSpec((TR, C), rows),
            pl.BlockSpec((TR, 1), rows),
            pl.BlockSpec((TR, 1), rows),
            pl.BlockSpec((TR, 1), rows),
            pl.BlockSpec((TR, E), rows),
            pl.BlockSpec((TR, 1), rows),
            pl.BlockSpec((1, 1), whole),
            pl.BlockSpec((TR, 1), rows),
            pl.BlockSpec((1, E), whole),
        ],
        out_shape=[
            jax.ShapeDtypeStruct((T, C), f32),
            jax.ShapeDtypeStruct((T, C), f32),
            jax.ShapeDtypeStruct((T, C), f32),
            jax.ShapeDtypeStruct((T, C), f32),
            jax.ShapeDtypeStruct((T, 1), jnp.int32),
            jax.ShapeDtypeStruct((T, 1), f32),
            jax.ShapeDtypeStruct((T, 1), f32),
            jax.ShapeDtypeStruct((T, E), f32),
            jax.ShapeDtypeStruct((T, 1), f32),
            jax.ShapeDtypeStruct((1, 1), f32),
            jax.ShapeDtypeStruct((T, 1), jnp.int32),
            jax.ShapeDtypeStruct((1, E), jnp.int32),
        ],
    )(x2d, xp2d, wrkvs, wo, wroute, confb, cap, g1, b1, g2, b2, wrec)


def _ffn_body(te_ref, h_ref, st_ref, x1_ref, sc_ref,
              w1_ref, b1_ref, w2_ref, b2_ref, ws1_ref,
              out_ref):
    i = pl.program_id(0)
    e = te_ref[i]
    h = h_ref[...]
    sel = (e == E - 1).astype(jnp.float32)
    base = (jnp.dot(h, w1_ref[0], preferred_element_type=jnp.float32)
            + sel * jnp.dot(st_ref[...], ws1_ref[...],
                            preferred_element_type=jnp.float32)
            + b1_ref[0])
    hr = jax.nn.relu(base)
    hid = jnp.where(e == E - 1, hr, hr * hr)
    out = jnp.dot(hid, w2_ref[0], preferred_element_type=jnp.float32) + b2_ref[0]
    out_ref[...] = x1_ref[...] + out * sc_ref[...]


def _ffn(tile_expert, h_s, st_s, x1_s, sc_s, w1, b1e, w2, b2e, ws1):
    rows = lambda i, te: (i, 0)
    byexp3 = lambda i, te: (te[i], 0, 0)
    whole = lambda i, te: (0, 0)
    f32 = jnp.float32
    grid_spec = pltpu.PrefetchScalarGridSpec(
        num_scalar_prefetch=1,
        grid=(NT,),
        in_specs=[
            pl.BlockSpec((TM, C), rows),
            pl.BlockSpec((TM, C), rows),
            pl.BlockSpec((TM, C), rows),
            pl.BlockSpec((TM, 1), rows),
            pl.BlockSpec((1, C, H), byexp3),
            pl.BlockSpec((1, 1, H), byexp3),
            pl.BlockSpec((1, H, C), byexp3),
            pl.BlockSpec((1, 1, C), byexp3),
            pl.BlockSpec((C, H), whole),
        ],
        out_specs=pl.BlockSpec((TM, C), rows),
        )
    return pl.pallas_call(
        _ffn_body,
        grid_spec=grid_spec,
        out_shape=jax.ShapeDtypeStruct((TPAD, C), f32),
    )(tile_expert, h_s, st_s, x1_s, sc_s, w1, b1e, w2, b2e, ws1)


def kernel(x, v_first, capital_shares, ln1_g, ln1_b, ln2_g, ln2_b, Wr, Wk, Wv,
           Wo, Ws, W1, b1, W2, b2, Ws1, Wrec, conf_w, conf_b, Wd, Wa):
    f32 = jnp.float32
    x2d = x.reshape(T, C)
    wrkvs = jnp.concatenate([Wr, Wk, Wv, Ws], axis=1)
    wroute = jnp.concatenate([conf_w.T, Wd, Wa], axis=1)

    (x1, h, st, v, win2, cost2, diff2, aff, scale2, rec_sum, rank2, cnt2) = \
        _prologue(
            x2d, x2d, wrkvs, Wo, wroute, conf_b.reshape(1, E),
            capital_shares.reshape(1, E), ln1_g.reshape(1, C),
            ln1_b.reshape(1, C), ln2_g.reshape(1, C), ln2_b.reshape(1, C), Wrec)

    winners = win2[:, 0]
    counts = cnt2[0]
    tiles_e = (counts + TM - 1) // TM
    cum_tiles = jnp.cumsum(tiles_e)
    pstart = (cum_tiles - tiles_e) * TM
    ti = jnp.arange(NT)
    tile_expert = jnp.minimum(
        jnp.sum((ti[:, None] >= cum_tiles[None, :]).astype(jnp.int32), axis=1),
        E - 1).astype(jnp.int32)
    inv_perm = (pstart[winners] + rank2[:, 0]).astype(jnp.int32)
    qi = jnp.arange(TPAD)
    packed = jnp.zeros((TPAD,), jnp.int32).at[inv_perm].set(
        jnp.arange(T, dtype=jnp.int32) + 1)
    src_row = jnp.where(packed > 0, packed - 1, qi % T).astype(jnp.int32)
    sc_s = scale2[src_row]

    h_s = _row_gather1(h, src_row, TPAD)
    st_s = _row_gather1(st, src_row, TPAD)
    x1_s = _row_gather1(x1, src_row, TPAD)

    ffn_out = _ffn(tile_expert, h_s, st_s, x1_s, sc_s,
                   W1, b1.reshape(E, 1, H), W2, b2.reshape(E, 1, C), Ws1)

    x_out = _row_gather1(ffn_out, inv_perm, T)

    cnt7 = counts[E - 1]
    recon = jnp.where(cnt7 > 0, rec_sum[0, 0] / (cnt7 * C).astype(f32), 0.0)

    return (x_out.reshape(1, T, C), v.reshape(1, T, C), winners.reshape(1, T),
            cost2[:, 0].reshape(1, T), diff2.reshape(1, T, 1),
            aff.reshape(1, T, E), recon)


# speedup vs baseline: 2.0115x; 1.0539x over previous
import functools

import jax
import jax.numpy as jnp
from jax import lax
from jax.experimental import pallas as pl
from jax.experimental.pallas import tpu as pltpu
from jax.experimental.pallas import tpu_sc as plsc

T = 2048
C = 768
E = 8
H = 4 * C
TM = 128
NT = T // TM + 8
TPAD = NT * TM
TR = 256


_NC, _NS = 2, 16
_NW = _NC * _NS


def _sc_mesh():
    return plsc.VectorSubcoreMesh(core_axis_name="c", subcore_axis_name="s",
                                  num_cores=_NC, num_subcores=_NS)


def _row_gather1(src, idx, n_out):
    bpw = n_out // _NW
    f32 = jnp.float32

    @functools.partial(
        pl.kernel, out_type=jax.ShapeDtypeStruct((n_out, C), f32),
        mesh=_sc_mesh(),
        scratch_types=[pltpu.VMEM((bpw,), jnp.int32),
                       pltpu.VMEM((bpw, C), f32),
                       pltpu.SemaphoreType.DMA])
    def k(src_hbm, idx_hbm, out_hbm, idx_v, rows_v, sem):
        wid = lax.axis_index("s") * _NC + lax.axis_index("c")
        base = wid * bpw
        pltpu.sync_copy(idx_hbm.at[pl.ds(base, bpw)], idx_v)
        pltpu.async_copy(src_hbm.at[idx_v], rows_v, sem).wait()
        pltpu.sync_copy(rows_v, out_hbm.at[pl.ds(base, bpw)])

    return k(src, idx)


def _ln(z, g, b):
    m = jnp.mean(z, axis=-1, keepdims=True)
    v = jnp.mean((z - m) ** 2, axis=-1, keepdims=True)
    return (z - m) * lax.rsqrt(v + 1e-5) * g + b


def _prologue_body(x_ref, xp_ref, wrkvs_ref, wo_ref, wroute_ref, confb_ref,
                   cap_ref, ln1g_ref, ln1b_ref, ln2g_ref, ln2b_ref, wrec_ref,
                   x1_ref, h_ref, st_ref, v_ref, win_ref, cost_ref, diff_ref,
                   aff_ref, scale_ref, rec_ref, rank_ref, cnt_ref):
    i = pl.program_id(0)
    g1, b1 = ln1g_ref[...], ln1b_ref[...]
    h1 = _ln(x_ref[...], g1, b1)
    h1p = _ln(xp_ref[TR - 1:TR, :], g1, b1)
    h1s = jnp.concatenate([h1p, h1[:TR - 1, :]], axis=0)
    row = lax.broadcasted_iota(jnp.int32, h1s.shape, 0) + i * TR
    h1s = jnp.where(row == 0, 0.0, h1s)
    mix = 0.5 * (h1 + h1s)
    rkvs = jnp.dot(mix, wrkvs_ref[...], preferred_element_type=jnp.float32)
    r = jax.nn.sigmoid(rkvs[:, 0:C])
    k = rkvs[:, C:2 * C]
    v = rkvs[:, 2 * C:3 * C]
    st = rkvs[:, 3 * C:4 * C]
    att = jnp.dot(r * k * v, wo_ref[...], preferred_element_type=jnp.float32)
    x1 = x_ref[...] + att
    h = _ln(x1, ln2g_ref[...], ln2b_ref[...])
    route = jnp.dot(h, wroute_ref[...], preferred_element_type=jnp.float32)
    conf = jax.nn.sigmoid(route[:, 0:E] + confb_ref[...])
    diff = jax.nn.sigmoid(route[:, E:E + 1])
    aff = route[:, E + 1:E + 1 + E]
    bids = conf * cap_ref[...] + 0.01 * aff
    maxb = jnp.max(bids, axis=-1, keepdims=True)
    eio = lax.broadcasted_iota(jnp.int32, bids.shape, 1)
    win = jnp.min(jnp.where(bids >= maxb, eio, E), axis=-1, keepdims=True)
    wb = jnp.sum(jnp.where(eio == win, conf, 0.0), axis=-1, keepdims=True)
    x1_ref[...] = x1
    h_ref[...] = h
    st_ref[...] = st
    v_ref[...] = v
    win_ref[...] = win
    cost_ref[...] = maxb * diff
    diff_ref[...] = diff
    aff_ref[...] = aff
    scale_ref[...] = wb / (wb + 1e-6)

    @pl.when(i == 0)
    def _init():
        rec_ref[...] = jnp.zeros_like(rec_ref)
        cnt_ref[...] = jnp.zeros_like(cnt_ref)

    rr = jnp.dot(h, wrec_ref[...], preferred_element_type=jnp.float32) - st
    m7 = (win == E - 1).astype(jnp.float32)
    rec_ref[...] += jnp.sum(
        jnp.sum(rr * rr, axis=-1, keepdims=True) * m7).reshape(1, 1)

    oh = (eio == win).astype(jnp.float32)
    rio = lax.broadcasted_iota(jnp.int32, (TR, TR), 0)
    cio = lax.broadcasted_iota(jnp.int32, (TR, TR), 1)
    tri = (rio > cio).astype(jnp.float32)
    excl = jnp.dot(tri, oh, preferred_element_type=jnp.float32)
    base = cnt_ref[...].astype(jnp.float32)
    rank_ref[...] = jnp.sum(oh * (excl + base), axis=1,
                            keepdims=True).astype(jnp.int32)
    cnt_ref[...] += jnp.sum(oh, axis=0, keepdims=True).astype(jnp.int32)


def _prologue(x2d, xp2d, wrkvs, wo, wroute, confb, cap, g1, b1, g2, b2, wrec):
    rows = lambda i: (i, 0)
    prev = lambda i: (jnp.maximum(i - 1, 0), 0)
    whole = lambda i: (0, 0)
    f32 = jnp.float32
    return pl.pallas_call(
        _prologue_body,
        grid=(T // TR,),
        in_specs=[
            pl.BlockSpec((TR, C), rows),
            pl.BlockSpec((TR, C), prev),
            pl.BlockSpec((C, 4 * C), whole),
            pl.BlockSpec((C, C), whole),
            pl.BlockSpec((C, 2 * E + 1), whole),
            pl.BlockSpec((1, E), whole),
            pl.BlockSpec((1, E), whole),
            pl.BlockSpec((1, C), whole),
            pl.BlockSpec((1, C), whole),
            pl.BlockSpec((1, C), whole),
            pl.BlockSpec((1, C), whole),
            pl.BlockSpec((C, C), whole),
        ],
        out_specs=[
            pl.BlockSpec((TR, C), rows),
            pl.BlockSpec((TR, C), rows),
            pl.BlockSpec((TR, C), rows),
            pl.BlockSpec((TR, C), rows),
            pl.BlockSpec((TR, 1), rows),
            pl.BlockSpec((TR, 1), rows),
            pl.BlockSpec((TR, 1), rows),
            pl.BlockSpec((TR, E), rows),
            pl.BlockSpec((TR, 1), rows),
            pl.BlockSpec((1, 1), whole),
            pl.BlockSpec((TR, 1), rows),
            pl.BlockSpec((1, E), whole),
        ],
        out_shape=[
            jax.ShapeDtypeStruct((T, C), f32),
            jax.ShapeDtypeStruct((T, C), f32),
            jax.ShapeDtypeStruct((T, C), f32),
            jax.ShapeDtypeStruct((T, C), f32),
            jax.ShapeDtypeStruct((T, 1), jnp.int32),
            jax.ShapeDtypeStruct((T, 1), f32),
            jax.ShapeDtypeStruct((T, 1), f32),
            jax.ShapeDtypeStruct((T, E), f32),
            jax.ShapeDtypeStruct((T, 1), f32),
            jax.ShapeDtypeStruct((1, 1), f32),
            jax.ShapeDtypeStruct((T, 1), jnp.int32),
            jax.ShapeDtypeStruct((1, E), jnp.int32),
        ],
    )(x2d, xp2d, wrkvs, wo, wroute, confb, cap, g1, b1, g2, b2, wrec)


def _ffn(tile_expert, h_s, st_s, x1_s, sc_s, w1, b1e, w2, b2e, ws1):
    f32 = jnp.float32
    look = pl.Buffered(buffer_count=2, use_lookahead=True)

    def inner(te_ref, h_hbm, st_hbm, x1_hbm, sc_hbm, w1_hbm, b1_hbm,
              w2_hbm, b2_hbm, ws1_ref, out_hbm):
        rows = lambda i: (i, 0)
        byexp3 = lambda i: (te_ref[i], 0, 0)

        def kbody(idx, h_ref, st_ref, x1_ref, sc_ref, w1_ref, b1_ref,
                  w2_ref, b2_ref, out_ref):
            i = idx[0]
            e = te_ref[i]
            h = h_ref[...]
            sel = (e == E - 1).astype(f32)
            base = (jnp.dot(h, w1_ref[0], preferred_element_type=f32)
                    + sel * jnp.dot(st_ref[...], ws1_ref[...],
                                    preferred_element_type=f32)
                    + b1_ref[0])
            hr = jax.nn.relu(base)
            hid = jnp.where(e == E - 1, hr, hr * hr)
            out = jnp.dot(hid, w2_ref[0], preferred_element_type=f32) + b2_ref[0]
            out_ref[...] = x1_ref[...] + out * sc_ref[...]

        pipeline = pltpu.emit_pipeline(
            kbody,
            grid=(NT,),
            in_specs=[
                pl.BlockSpec((TM, C), rows),
                pl.BlockSpec((TM, C), rows),
                pl.BlockSpec((TM, C), rows),
                pl.BlockSpec((TM, 1), rows),
                pl.BlockSpec((1, C, H), byexp3, pipeline_mode=look),
                pl.BlockSpec((1, 1, H), byexp3, pipeline_mode=look),
                pl.BlockSpec((1, H, C), byexp3, pipeline_mode=look),
                pl.BlockSpec((1, 1, C), byexp3, pipeline_mode=look),
            ],
            out_specs=[pl.BlockSpec((TM, C), rows)],
            _explicit_indices=True,
        )
        pipeline(h_hbm, st_hbm, x1_hbm, sc_hbm, w1_hbm, b1_hbm, w2_hbm,
                 b2_hbm, out_hbm)

    anyspace = pl.BlockSpec(memory_space=pl.ANY)
    return pl.pallas_call(
        inner,
        in_specs=[
            pl.BlockSpec(memory_space=pltpu.SMEM),
            anyspace, anyspace, anyspace, anyspace, anyspace, anyspace,
            anyspace, anyspace,
            pl.BlockSpec(memory_space=pltpu.VMEM),
        ],
        out_specs=anyspace,
        out_shape=jax.ShapeDtypeStruct((TPAD, C), f32),
    )(tile_expert, h_s, st_s, x1_s, sc_s, w1, b1e, w2, b2e, ws1)


def kernel(x, v_first, capital_shares, ln1_g, ln1_b, ln2_g, ln2_b, Wr, Wk, Wv,
           Wo, Ws, W1, b1, W2, b2, Ws1, Wrec, conf_w, conf_b, Wd, Wa):
    f32 = jnp.float32
    x2d = x.reshape(T, C)
    wrkvs = jnp.concatenate([Wr, Wk, Wv, Ws], axis=1)
    wroute = jnp.concatenate([conf_w.T, Wd, Wa], axis=1)

    (x1, h, st, v, win2, cost2, diff2, aff, scale2, rec_sum, rank2, cnt2) = \
        _prologue(
            x2d, x2d, wrkvs, Wo, wroute, conf_b.reshape(1, E),
            capital_shares.reshape(1, E), ln1_g.reshape(1, C),
            ln1_b.reshape(1, C), ln2_g.reshape(1, C), ln2_b.reshape(1, C), Wrec)

    winners = win2[:, 0]
    counts = cnt2[0]
    tiles_e = (counts + TM - 1) // TM
    cum_tiles = jnp.cumsum(tiles_e)
    pstart = (cum_tiles - tiles_e) * TM
    ti = jnp.arange(NT)
    tile_expert = jnp.minimum(
        jnp.sum((ti[:, None] >= cum_tiles[None, :]).astype(jnp.int32), axis=1),
        E - 1).astype(jnp.int32)
    inv_perm = (pstart[winners] + rank2[:, 0]).astype(jnp.int32)
    qi = jnp.arange(TPAD)
    packed = jnp.zeros((TPAD,), jnp.int32).at[inv_perm].set(
        jnp.arange(T, dtype=jnp.int32) + 1)
    src_row = jnp.where(packed > 0, packed - 1, qi % T).astype(jnp.int32)
    sc_s = scale2[src_row]

    h_s = _row_gather1(h, src_row, TPAD)
    st_s = _row_gather1(st, src_row, TPAD)
    x1_s = _row_gather1(x1, src_row, TPAD)

    ffn_out = _ffn(tile_expert, h_s, st_s, x1_s, sc_s,
                   W1, b1.reshape(E, 1, H), W2, b2.reshape(E, 1, C), Ws1)

    x_out = _row_gather1(ffn_out, inv_perm, T)

    cnt7 = counts[E - 1]
    recon = jnp.where(cnt7 > 0, rec_sum[0, 0] / (cnt7 * C).astype(f32), 0.0)

    return (x_out.reshape(1, T, C), v.reshape(1, T, C), winners.reshape(1, T),
            cost2[:, 0].reshape(1, T), diff2.reshape(1, T, 1),
            aff.reshape(1, T, E), recon)


# speedup vs baseline: 2.1717x; 1.0797x over previous
import functools

import jax
import jax.numpy as jnp
from jax import lax
from jax.experimental import pallas as pl
from jax.experimental.pallas import tpu as pltpu
from jax.experimental.pallas import tpu_sc as plsc

T = 2048
C = 768
E = 8
H = 4 * C
TM = 128
NT = T // TM + 8
TPAD = NT * TM
TR = 256


_NC, _NS = 2, 16
_NW = _NC * _NS


def _sc_mesh():
    return plsc.VectorSubcoreMesh(core_axis_name="c", subcore_axis_name="s",
                                  num_cores=_NC, num_subcores=_NS)


def _row_gather3(h, st, x1, idx):
    bpw = TPAD // _NW
    ck = bpw // 2
    f32 = jnp.float32

    @functools.partial(
        pl.kernel, out_type=[jax.ShapeDtypeStruct((TPAD, C), f32)] * 3,
        mesh=_sc_mesh(),
        scratch_types=[pltpu.VMEM((bpw,), jnp.int32),
                       pltpu.VMEM((ck, C), f32),
                       pltpu.VMEM((ck, C), f32),
                       pltpu.VMEM((ck, C), f32),
                       pltpu.SemaphoreType.DMA,
                       pltpu.SemaphoreType.DMA,
                       pltpu.SemaphoreType.DMA])
    def k(h_hbm, st_hbm, x1_hbm, idx_hbm, oh_hbm, ost_hbm, ox1_hbm,
          idx_v, bh, bst, bx1, sem0, sem1, sem2):
        wid = lax.axis_index("s") * _NC + lax.axis_index("c")
        base = wid * bpw
        pltpu.sync_copy(idx_hbm.at[pl.ds(base, bpw)], idx_v)
        for c in range(2):
            off = base + c * ck
            idx_c = idx_v.at[pl.ds(c * ck, ck)]
            a0 = pltpu.async_copy(h_hbm.at[idx_c], bh, sem0)
            a1 = pltpu.async_copy(st_hbm.at[idx_c], bst, sem1)
            a2 = pltpu.async_copy(x1_hbm.at[idx_c], bx1, sem2)
            a0.wait()
            pltpu.sync_copy(bh, oh_hbm.at[pl.ds(off, ck)])
            a1.wait()
            pltpu.sync_copy(bst, ost_hbm.at[pl.ds(off, ck)])
            a2.wait()
            pltpu.sync_copy(bx1, ox1_hbm.at[pl.ds(off, ck)])

    return k(h, st, x1, idx)


def _row_gather1(src, idx, n_out):
    bpw = n_out // _NW
    f32 = jnp.float32

    @functools.partial(
        pl.kernel, out_type=jax.ShapeDtypeStruct((n_out, C), f32),
        mesh=_sc_mesh(),
        scratch_types=[pltpu.VMEM((bpw,), jnp.int32),
                       pltpu.VMEM((bpw, C), f32),
                       pltpu.SemaphoreType.DMA])
    def k(src_hbm, idx_hbm, out_hbm, idx_v, rows_v, sem):
        wid = lax.axis_index("s") * _NC + lax.axis_index("c")
        base = wid * bpw
        pltpu.sync_copy(idx_hbm.at[pl.ds(base, bpw)], idx_v)
        pltpu.async_copy(src_hbm.at[idx_v], rows_v, sem).wait()
        pltpu.sync_copy(rows_v, out_hbm.at[pl.ds(base, bpw)])

    return k(src, idx)


def _ln(z, g, b):
    m = jnp.mean(z, axis=-1, keepdims=True)
    v = jnp.mean((z - m) ** 2, axis=-1, keepdims=True)
    return (z - m) * lax.rsqrt(v + 1e-5) * g + b


def _prologue_body(x_ref, xp_ref, wr_ref, wk_ref, wv_ref, ws_ref, wo_ref,
                   wroute_ref, confb_ref,
                   cap_ref, ln1g_ref, ln1b_ref, ln2g_ref, ln2b_ref, wrec_ref,
                   x1_ref, h_ref, st_ref, v_ref, win_ref, cost_ref, diff_ref,
                   aff_ref, scale_ref, rec_ref, rank_ref, cnt_ref):
    i = pl.program_id(0)
    g1, b1 = ln1g_ref[...], ln1b_ref[...]
    h1 = _ln(x_ref[...], g1, b1)
    h1p = _ln(xp_ref[TR - 1:TR, :], g1, b1)
    h1s = jnp.concatenate([h1p, h1[:TR - 1, :]], axis=0)
    row = lax.broadcasted_iota(jnp.int32, h1s.shape, 0) + i * TR
    h1s = jnp.where(row == 0, 0.0, h1s)
    mix = 0.5 * (h1 + h1s)
    r = jax.nn.sigmoid(jnp.dot(mix, wr_ref[...],
                               preferred_element_type=jnp.float32))
    k = jnp.dot(mix, wk_ref[...], preferred_element_type=jnp.float32)
    v = jnp.dot(mix, wv_ref[...], preferred_element_type=jnp.float32)
    st = jnp.dot(mix, ws_ref[...], preferred_element_type=jnp.float32)
    att = jnp.dot(r * k * v, wo_ref[...], preferred_element_type=jnp.float32)
    x1 = x_ref[...] + att
    h = _ln(x1, ln2g_ref[...], ln2b_ref[...])
    route = jnp.dot(h, wroute_ref[...], preferred_element_type=jnp.float32)
    conf = jax.nn.sigmoid(route[:, 0:E] + confb_ref[...])
    diff = jax.nn.sigmoid(route[:, E:E + 1])
    aff = route[:, E + 1:E + 1 + E]
    bids = conf * cap_ref[...] + 0.01 * aff
    maxb = jnp.max(bids, axis=-1, keepdims=True)
    eio = lax.broadcasted_iota(jnp.int32, bids.shape, 1)
    win = jnp.min(jnp.where(bids >= maxb, eio, E), axis=-1, keepdims=True)
    wb = jnp.sum(jnp.where(eio == win, conf, 0.0), axis=-1, keepdims=True)
    x1_ref[...] = x1
    h_ref[...] = h
    st_ref[...] = st
    v_ref[...] = v
    win_ref[...] = win
    cost_ref[...] = maxb * diff
    diff_ref[...] = diff
    aff_ref[...] = aff
    scale_ref[...] = wb / (wb + 1e-6)

    @pl.when(i == 0)
    def _init():
        rec_ref[...] = jnp.zeros_like(rec_ref)
        cnt_ref[...] = jnp.zeros_like(cnt_ref)

    rr = jnp.dot(h, wrec_ref[...], preferred_element_type=jnp.float32) - st
    m7 = (win == E - 1).astype(jnp.float32)
    rec_ref[...] += jnp.sum(
        jnp.sum(rr * rr, axis=-1, keepdims=True) * m7).reshape(1, 1)

    oh = (eio == win).astype(jnp.float32)
    rio = lax.broadcasted_iota(jnp.int32, (TR, TR), 0)
    cio = lax.broadcasted_iota(jnp.int32, (TR, TR), 1)
    tri = (rio > cio).astype(jnp.float32)
    excl = jnp.dot(tri, oh, preferred_element_type=jnp.float32)
    base = cnt_ref[...].astype(jnp.float32)
    rank_ref[...] = jnp.sum(oh * (excl + base), axis=1,
                            keepdims=True).astype(jnp.int32)
    cnt_ref[...] += jnp.sum(oh, axis=0, keepdims=True).astype(jnp.int32)


def _prologue(x2d, xp2d, wr, wk, wv, ws, wo, wroute, confb, cap, g1, b1, g2,
              b2, wrec):
    rows = lambda i: (i, 0)
    prev = lambda i: (jnp.maximum(i - 1, 0), 0)
    whole = lambda i: (0, 0)
    f32 = jnp.float32
    return pl.pallas_call(
        _prologue_body,
        grid=(T // TR,),
        in_specs=[
            pl.BlockSpec((TR, C), rows),
            pl.BlockSpec((TR, C), prev),
            pl.BlockSpec((C, C), whole),
            pl.BlockSpec((C, C), whole),
            pl.BlockSpec((C, C), whole),
            pl.BlockSpec((C, C), whole),
            pl.BlockSpec((C, C), whole),
            pl.BlockSpec((C, 2 * E + 1), whole),
            pl.BlockSpec((1, E), whole),
            pl.BlockSpec((1, E), whole),
            pl.BlockSpec((1, C), whole),
            pl.BlockSpec((1, C), whole),
            pl.BlockSpec((1, C), whole),
            pl.BlockSpec((1, C), whole),
            pl.BlockSpec((C, C), whole),
        ],
        out_specs=[
            pl.BlockSpec((TR, C), rows),
            pl.BlockSpec((TR, C), rows),
            pl.BlockSpec((TR, C), rows),
            pl.BlockSpec((TR, C), rows),
            pl.BlockSpec((TR, 1), rows),
            pl.BlockSpec((TR, 1), rows),
            pl.BlockSpec((TR, 1), rows),
            pl.BlockSpec((TR, E), rows),
            pl.BlockSpec((TR, 1), rows),
            pl.BlockSpec((1, 1), whole),
            pl.BlockSpec((TR, 1), rows),
            pl.BlockSpec((1, E), whole),
        ],
        out_shape=[
            jax.ShapeDtypeStruct((T, C), f32),
            jax.ShapeDtypeStruct((T, C), f32),
            jax.ShapeDtypeStruct((T, C), f32),
            jax.ShapeDtypeStruct((T, C), f32),
            jax.ShapeDtypeStruct((T, 1), jnp.int32),
            jax.ShapeDtypeStruct((T, 1), f32),
            jax.ShapeDtypeStruct((T, 1), f32),
            jax.ShapeDtypeStruct((T, E), f32),
            jax.ShapeDtypeStruct((T, 1), f32),
            jax.ShapeDtypeStruct((1, 1), f32),
            jax.ShapeDtypeStruct((T, 1), jnp.int32),
            jax.ShapeDtypeStruct((1, E), jnp.int32),
        ],
    )(x2d, xp2d, wr, wk, wv, ws, wo, wroute, confb, cap, g1, b1, g2, b2, wrec)


def _ffn(tile_expert, h_s, st_s, x1_s, sc_s, w1, b1e, w2, b2e, ws1):
    f32 = jnp.float32
    look = pl.Buffered(buffer_count=2, use_lookahead=True)

    def inner(te_ref, h_hbm, st_hbm, x1_hbm, sc_hbm, w1_hbm, b1_hbm,
              w2_hbm, b2_hbm, ws1_ref, out_hbm):
        rows = lambda i: (i, 0)
        byexp3 = lambda i: (te_ref[i], 0, 0)
        strows = lambda i: (jnp.where(te_ref[i] == E - 1, i, 0), 0)

        def kbody(idx, h_ref, st_ref, x1_ref, sc_ref, w1_ref, b1_ref,
                  w2_ref, b2_ref, out_ref):
            i = idx[0]
            e = te_ref[i]
            h = h_ref[...]
            sel = (e == E - 1).astype(f32)
            base = (jnp.dot(h, w1_ref[0], preferred_element_type=f32)
                    + sel * jnp.dot(st_ref[...], ws1_ref[...],
                                    preferred_element_type=f32)
                    + b1_ref[0])
            hr = jax.nn.relu(base)
            hid = jnp.where(e == E - 1, hr, hr * hr)
            out = jnp.dot(hid, w2_ref[0], preferred_element_type=f32) + b2_ref[0]
            out_ref[...] = x1_ref[...] + out * sc_ref[...]

        pipeline = pltpu.emit_pipeline(
            kbody,
            grid=(NT,),
            in_specs=[
                pl.BlockSpec((TM, C), rows),
                pl.BlockSpec((TM, C), strows),
                pl.BlockSpec((TM, C), rows),
                pl.BlockSpec((TM, 1), rows),
                pl.BlockSpec((1, C, H), byexp3, pipeline_mode=look),
                pl.BlockSpec((1, 1, H), byexp3, pipeline_mode=look),
                pl.BlockSpec((1, H, C), byexp3, pipeline_mode=look),
                pl.BlockSpec((1, 1, C), byexp3, pipeline_mode=look),
            ],
            out_specs=[pl.BlockSpec((TM, C), rows)],
            _explicit_indices=True,
        )
        pipeline(h_hbm, st_hbm, x1_hbm, sc_hbm, w1_hbm, b1_hbm, w2_hbm,
                 b2_hbm, out_hbm)

    anyspace = pl.BlockSpec(memory_space=pl.ANY)
    return pl.pallas_call(
        inner,
        in_specs=[
            pl.BlockSpec(memory_space=pltpu.SMEM),
            anyspace, anyspace, anyspace, anyspace, anyspace, anyspace,
            anyspace, anyspace,
            pl.BlockSpec(memory_space=pltpu.VMEM),
        ],
        out_specs=anyspace,
        out_shape=jax.ShapeDtypeStruct((TPAD, C), f32),
    )(tile_expert, h_s, st_s, x1_s, sc_s, w1, b1e, w2, b2e, ws1)


def kernel(x, v_first, capital_shares, ln1_g, ln1_b, ln2_g, ln2_b, Wr, Wk, Wv,
           Wo, Ws, W1, b1, W2, b2, Ws1, Wrec, conf_w, conf_b, Wd, Wa):
    f32 = jnp.float32
    x2d = x.reshape(T, C)
    wroute = jnp.concatenate([conf_w.T, Wd, Wa], axis=1)

    (x1, h, st, v, win2, cost2, diff2, aff, scale2, rec_sum, rank2, cnt2) = \
        _prologue(
            x2d, x2d, Wr, Wk, Wv, Ws, Wo, wroute, conf_b.reshape(1, E),
            capital_shares.reshape(1, E), ln1_g.reshape(1, C),
            ln1_b.reshape(1, C), ln2_g.reshape(1, C), ln2_b.reshape(1, C), Wrec)

    winners = win2[:, 0]
    counts = cnt2[0]
    tiles_e = (counts + TM - 1) // TM
    cum_tiles = jnp.cumsum(tiles_e)
    pstart = (cum_tiles - tiles_e) * TM
    ti = jnp.arange(NT)
    tile_expert = jnp.minimum(
        jnp.sum((ti[:, None] >= cum_tiles[None, :]).astype(jnp.int32), axis=1),
        E - 1).astype(jnp.int32)
    inv_perm = (pstart[winners] + rank2[:, 0]).astype(jnp.int32)
    qi = jnp.arange(TPAD)
    packed = jnp.zeros((TPAD,), jnp.int32).at[inv_perm].set(
        jnp.arange(T, dtype=jnp.int32) + 1)
    src_row = jnp.where(packed > 0, packed - 1, qi % T).astype(jnp.int32)
    sc_s = scale2[src_row]

    h_s, st_s, x1_s = _row_gather3(h, st, x1, src_row)

    ffn_out = _ffn(tile_expert, h_s, st_s, x1_s, sc_s,
                   W1, b1.reshape(E, 1, H), W2, b2.reshape(E, 1, C), Ws1)

    x_out = _row_gather1(ffn_out, inv_perm, T)

    cnt7 = counts[E - 1]
    recon = jnp.where(cnt7 > 0, rec_sum[0, 0] / (cnt7 * C).astype(f32), 0.0)

    return (x_out.reshape(1, T, C), v.reshape(1, T, C), winners.reshape(1, T),
            cost2[:, 0].reshape(1, T), diff2.reshape(1, T, 1),
            aff.reshape(1, T, E), recon)


# speedup vs baseline: 2.2024x; 1.0141x over previous
import functools

import jax
import jax.numpy as jnp
from jax import lax
from jax.experimental import pallas as pl
from jax.experimental.pallas import tpu as pltpu
from jax.experimental.pallas import tpu_sc as plsc

T = 2048
C = 768
E = 8
H = 4 * C
TM = 128
NT = T // TM + 8
TPAD = NT * TM
TR = 256


_NC, _NS = 2, 16
_NW = _NC * _NS


def _sc_mesh():
    return plsc.VectorSubcoreMesh(core_axis_name="c", subcore_axis_name="s",
                                  num_cores=_NC, num_subcores=_NS)


def _row_gather2(h, st, idx):
    bpw = TPAD // _NW
    ck = bpw // 2
    f32 = jnp.float32

    @functools.partial(
        pl.kernel, out_type=[jax.ShapeDtypeStruct((TPAD, C), f32)] * 2,
        mesh=_sc_mesh(),
        scratch_types=[pltpu.VMEM((bpw,), jnp.int32),
                       pltpu.VMEM((ck, C), f32),
                       pltpu.VMEM((ck, C), f32),
                       pltpu.SemaphoreType.DMA,
                       pltpu.SemaphoreType.DMA])
    def k(h_hbm, st_hbm, idx_hbm, oh_hbm, ost_hbm,
          idx_v, bh, bst, sem0, sem1):
        wid = lax.axis_index("s") * _NC + lax.axis_index("c")
        base = wid * bpw
        pltpu.sync_copy(idx_hbm.at[pl.ds(base, bpw)], idx_v)
        for c in range(2):
            off = base + c * ck
            idx_c = idx_v.at[pl.ds(c * ck, ck)]
            a0 = pltpu.async_copy(h_hbm.at[idx_c], bh, sem0)
            a1 = pltpu.async_copy(st_hbm.at[idx_c], bst, sem1)
            a0.wait()
            pltpu.sync_copy(bh, oh_hbm.at[pl.ds(off, ck)])
            a1.wait()
            pltpu.sync_copy(bst, ost_hbm.at[pl.ds(off, ck)])

    return k(h, st, idx)


def _combine_add(moe, x1, idx):
    bpw = T // _NW
    f32 = jnp.float32

    @functools.partial(
        pl.kernel, out_type=jax.ShapeDtypeStruct((T, C), f32),
        mesh=_sc_mesh(),
        scratch_types=[pltpu.VMEM((bpw,), jnp.int32),
                       pltpu.VMEM((bpw, C), f32),
                       pltpu.VMEM((bpw, C), f32),
                       pltpu.SemaphoreType.DMA])
    def k(moe_hbm, x1_hbm, idx_hbm, out_hbm, idx_v, mv, xv, sem):
        wid = lax.axis_index("s") * _NC + lax.axis_index("c")
        base = wid * bpw
        pltpu.sync_copy(idx_hbm.at[pl.ds(base, bpw)], idx_v)
        a = pltpu.async_copy(moe_hbm.at[idx_v], mv, sem)
        pltpu.sync_copy(x1_hbm.at[pl.ds(base, bpw)], xv)
        a.wait()

        def row(r, carry):
            for cc in range(C // 16):
                sl = pl.ds(cc * 16, 16)
                mv[r, sl] = mv[r, sl] + xv[r, sl]
            return carry

        lax.fori_loop(0, bpw, row, 0)
        pltpu.sync_copy(mv, out_hbm.at[pl.ds(base, bpw)])

    return k(moe, x1, idx)


def _row_gather1(src, idx, n_out):
    bpw = n_out // _NW
    f32 = jnp.float32

    @functools.partial(
        pl.kernel, out_type=jax.ShapeDtypeStruct((n_out, C), f32),
        mesh=_sc_mesh(),
        scratch_types=[pltpu.VMEM((bpw,), jnp.int32),
                       pltpu.VMEM((bpw, C), f32),
                       pltpu.SemaphoreType.DMA])
    def k(src_hbm, idx_hbm, out_hbm, idx_v, rows_v, sem):
        wid = lax.axis_index("s") * _NC + lax.axis_index("c")
        base = wid * bpw
        pltpu.sync_copy(idx_hbm.at[pl.ds(base, bpw)], idx_v)
        pltpu.async_copy(src_hbm.at[idx_v], rows_v, sem).wait()
        pltpu.sync_copy(rows_v, out_hbm.at[pl.ds(base, bpw)])

    return k(src, idx)


def _ln(z, g, b):
    m = jnp.mean(z, axis=-1, keepdims=True)
    v = jnp.mean((z - m) ** 2, axis=-1, keepdims=True)
    return (z - m) * lax.rsqrt(v + 1e-5) * g + b


def _prologue_body(x_ref, xp_ref, wr_ref, wk_ref, wv_ref, ws_ref, wo_ref,
                   wroute_ref, confb_ref,
                   cap_ref, ln1g_ref, ln1b_ref, ln2g_ref, ln2b_ref, wrec_ref,
                   x1_ref, h_ref, st_ref, v_ref, win_ref, cost_ref, diff_ref,
                   aff_ref, scale_ref, rec_ref, rank_ref, cnt_ref):
    i = pl.program_id(0)
    g1, b1 = ln1g_ref[...], ln1b_ref[...]
    h1 = _ln(x_ref[...], g1, b1)
    h1p = _ln(xp_ref[TR - 1:TR, :], g1, b1)
    h1s = jnp.concatenate([h1p, h1[:TR - 1, :]], axis=0)
    row = lax.broadcasted_iota(jnp.int32, h1s.shape, 0) + i * TR
    h1s = jnp.where(row == 0, 0.0, h1s)
    mix = 0.5 * (h1 + h1s)
    r = jax.nn.sigmoid(jnp.dot(mix, wr_ref[...],
                               preferred_element_type=jnp.float32))
    k = jnp.dot(mix, wk_ref[...], preferred_element_type=jnp.float32)
    v = jnp.dot(mix, wv_ref[...], preferred_element_type=jnp.float32)
    st = jnp.dot(mix, ws_ref[...], preferred_element_type=jnp.float32)
    att = jnp.dot(r * k * v, wo_ref[...], preferred_element_type=jnp.float32)
    x1 = x_ref[...] + att
    h = _ln(x1, ln2g_ref[...], ln2b_ref[...])
    route = jnp.dot(h, wroute_ref[...], preferred_element_type=jnp.float32)
    conf = jax.nn.sigmoid(route[:, 0:E] + confb_ref[...])
    diff = jax.nn.sigmoid(route[:, E:E + 1])
    aff = route[:, E + 1:E + 1 + E]
    bids = conf * cap_ref[...] + 0.01 * aff
    maxb = jnp.max(bids, axis=-1, keepdims=True)
    eio = lax.broadcasted_iota(jnp.int32, bids.shape, 1)
    win = jnp.min(jnp.where(bids >= maxb, eio, E), axis=-1, keepdims=True)
    wb = jnp.sum(jnp.where(eio == win, conf, 0.0), axis=-1, keepdims=True)
    x1_ref[...] = x1
    h_ref[...] = h
    st_ref[...] = st
    v_ref[...] = v
    win_ref[...] = win
    cost_ref[...] = maxb * diff
    diff_ref[...] = diff
    aff_ref[...] = aff
    scale_ref[...] = wb / (wb + 1e-6)

    @pl.when(i == 0)
    def _init():
        rec_ref[...] = jnp.zeros_like(rec_ref)
        cnt_ref[...] = jnp.zeros_like(cnt_ref)

    rr = jnp.dot(h, wrec_ref[...], preferred_element_type=jnp.float32) - st
    m7 = (win == E - 1).astype(jnp.float32)
    rec_ref[...] += jnp.sum(
        jnp.sum(rr * rr, axis=-1, keepdims=True) * m7).reshape(1, 1)

    oh = (eio == win).astype(jnp.float32)
    rio = lax.broadcasted_iota(jnp.int32, (TR, TR), 0)
    cio = lax.broadcasted_iota(jnp.int32, (TR, TR), 1)
    tri = (rio > cio).astype(jnp.float32)
    excl = jnp.dot(tri, oh, preferred_element_type=jnp.float32)
    base = cnt_ref[...].astype(jnp.float32)
    rank_ref[...] = jnp.sum(oh * (excl + base), axis=1,
                            keepdims=True).astype(jnp.int32)
    cnt_ref[...] += jnp.sum(oh, axis=0, keepdims=True).astype(jnp.int32)


def _prologue(x2d, xp2d, wr, wk, wv, ws, wo, wroute, confb, cap, g1, b1, g2,
              b2, wrec):
    rows = lambda i: (i, 0)
    prev = lambda i: (jnp.maximum(i - 1, 0), 0)
    whole = lambda i: (0, 0)
    f32 = jnp.float32
    return pl.pallas_call(
        _prologue_body,
        grid=(T // TR,),
        in_specs=[
            pl.BlockSpec((TR, C), rows),
            pl.BlockSpec((TR, C), prev),
            pl.BlockSpec((C, C), whole),
            pl.BlockSpec((C, C), whole),
            pl.BlockSpec((C, C), whole),
            pl.BlockSpec((C, C), whole),
            pl.BlockSpec((C, C), whole),
            pl.BlockSpec((C, 2 * E + 1), whole),
            pl.BlockSpec((1, E), whole),
            pl.BlockSpec((1, E), whole),
            pl.BlockSpec((1, C), whole),
            pl.BlockSpec((1, C), whole),
            pl.BlockSpec((1, C), whole),
            pl.BlockSpec((1, C), whole),
            pl.BlockSpec((C, C), whole),
        ],
        out_specs=[
            pl.BlockSpec((TR, C), rows),
            pl.BlockSpec((TR, C), rows),
            pl.BlockSpec((TR, C), rows),
            pl.BlockSpec((TR, C), rows),
            pl.BlockSpec((TR, 1), rows),
            pl.BlockSpec((TR, 1), rows),
            pl.BlockSpec((TR, 1), rows),
            pl.BlockSpec((TR, E), rows),
            pl.BlockSpec((TR, 1), rows),
            pl.BlockSpec((1, 1), whole),
            pl.BlockSpec((TR, 1), rows),
            pl.BlockSpec((1, E), whole),
        ],
        out_shape=[
            jax.ShapeDtypeStruct((T, C), f32),
            jax.ShapeDtypeStruct((T, C), f32),
            jax.ShapeDtypeStruct((T, C), f32),
            jax.ShapeDtypeStruct((T, C), f32),
            jax.ShapeDtypeStruct((T, 1), jnp.int32),
            jax.ShapeDtypeStruct((T, 1), f32),
            jax.ShapeDtypeStruct((T, 1), f32),
            jax.ShapeDtypeStruct((T, E), f32),
            jax.ShapeDtypeStruct((T, 1), f32),
            jax.ShapeDtypeStruct((1, 1), f32),
            jax.ShapeDtypeStruct((T, 1), jnp.int32),
            jax.ShapeDtypeStruct((1, E), jnp.int32),
        ],
    )(x2d, xp2d, wr, wk, wv, ws, wo, wroute, confb, cap, g1, b1, g2, b2, wrec)


def _ffn(tile_expert, h_s, st_s, sc_s, w1, b1e, w2, b2e, ws1):
    f32 = jnp.float32
    look = pl.Buffered(buffer_count=2, use_lookahead=True)

    def inner(te_ref, h_hbm, st_hbm, sc_hbm, w1_hbm, b1_hbm,
              w2_hbm, b2_hbm, ws1_ref, out_hbm):
        rows = lambda i: (i, 0)
        byexp3 = lambda i: (te_ref[i], 0, 0)
        strows = lambda i: (jnp.where(te_ref[i] == E - 1, i, 0), 0)

        def kbody(idx, h_ref, st_ref, sc_ref, w1_ref, b1_ref,
                  w2_ref, b2_ref, out_ref):
            i = idx[0]
            e = te_ref[i]
            h = h_ref[...]
            sel = (e == E - 1).astype(f32)
            base = (jnp.dot(h, w1_ref[0], preferred_element_type=f32)
                    + sel * jnp.dot(st_ref[...], ws1_ref[...],
                                    preferred_element_type=f32)
                    + b1_ref[0])
            hr = jax.nn.relu(base)
            hid = jnp.where(e == E - 1, hr, hr * hr)
            out = jnp.dot(hid, w2_ref[0], preferred_element_type=f32) + b2_ref[0]
            out_ref[...] = out * sc_ref[...]

        pipeline = pltpu.emit_pipeline(
            kbody,
            grid=(NT,),
            in_specs=[
                pl.BlockSpec((TM, C), rows),
                pl.BlockSpec((TM, C), strows),
                pl.BlockSpec((TM, 1), rows),
                pl.BlockSpec((1, C, H), byexp3, pipeline_mode=look),
                pl.BlockSpec((1, 1, H), byexp3, pipeline_mode=look),
                pl.BlockSpec((1, H, C), byexp3, pipeline_mode=look),
                pl.BlockSpec((1, 1, C), byexp3, pipeline_mode=look),
            ],
            out_specs=[pl.BlockSpec((TM, C), rows)],
            _explicit_indices=True,
        )
        pipeline(h_hbm, st_hbm, sc_hbm, w1_hbm, b1_hbm, w2_hbm,
                 b2_hbm, out_hbm)

    anyspace = pl.BlockSpec(memory_space=pl.ANY)
    return pl.pallas_call(
        inner,
        in_specs=[
            pl.BlockSpec(memory_space=pltpu.SMEM),
            anyspace, anyspace, anyspace, anyspace, anyspace, anyspace,
            anyspace,
            pl.BlockSpec(memory_space=pltpu.VMEM),
        ],
        out_specs=anyspace,
        out_shape=jax.ShapeDtypeStruct((TPAD, C), f32),
    )(tile_expert, h_s, st_s, sc_s, w1, b1e, w2, b2e, ws1)


def kernel(x, v_first, capital_shares, ln1_g, ln1_b, ln2_g, ln2_b, Wr, Wk, Wv,
           Wo, Ws, W1, b1, W2, b2, Ws1, Wrec, conf_w, conf_b, Wd, Wa):
    f32 = jnp.float32
    x2d = x.reshape(T, C)
    wroute = jnp.concatenate([conf_w.T, Wd, Wa], axis=1)

    (x1, h, st, v, win2, cost2, diff2, aff, scale2, rec_sum, rank2, cnt2) = \
        _prologue(
            x2d, x2d, Wr, Wk, Wv, Ws, Wo, wroute, conf_b.reshape(1, E),
            capital_shares.reshape(1, E), ln1_g.reshape(1, C),
            ln1_b.reshape(1, C), ln2_g.reshape(1, C), ln2_b.reshape(1, C), Wrec)

    winners = win2[:, 0]
    counts = cnt2[0]
    tiles_e = (counts + TM - 1) // TM
    cum_tiles = jnp.cumsum(tiles_e)
    pstart = (cum_tiles - tiles_e) * TM
    ti = jnp.arange(NT)
    tile_expert = jnp.minimum(
        jnp.sum((ti[:, None] >= cum_tiles[None, :]).astype(jnp.int32), axis=1),
        E - 1).astype(jnp.int32)
    inv_perm = (pstart[winners] + rank2[:, 0]).astype(jnp.int32)
    qi = jnp.arange(TPAD)
    packed = jnp.zeros((TPAD,), jnp.int32).at[inv_perm].set(
        jnp.arange(T, dtype=jnp.int32) + 1)
    src_row = jnp.where(packed > 0, packed - 1, qi % T).astype(jnp.int32)
    sc_s = scale2[src_row]

    h_s, st_s = _row_gather2(h, st, src_row)

    ffn_out = _ffn(tile_expert, h_s, st_s, sc_s,
                   W1, b1.reshape(E, 1, H), W2, b2.reshape(E, 1, C), Ws1)

    x_out = _combine_add(ffn_out, x1, inv_perm)

    cnt7 = counts[E - 1]
    recon = jnp.where(cnt7 > 0, rec_sum[0, 0] / (cnt7 * C).astype(f32), 0.0)

    return (x_out.reshape(1, T, C), v.reshape(1, T, C), winners.reshape(1, T),
            cost2[:, 0].reshape(1, T), diff2.reshape(1, T, 1),
            aff.reshape(1, T, E), recon)


# speedup vs baseline: 2.2135x; 1.0050x over previous
import functools

import jax
import jax.numpy as jnp
from jax import lax
from jax.experimental import pallas as pl
from jax.experimental.pallas import tpu as pltpu
from jax.experimental.pallas import tpu_sc as plsc

T = 2048
C = 768
E = 8
H = 4 * C
TM = 128
NT = T // TM + 8
TPAD = NT * TM
TR = 256


_NC, _NS = 2, 16
_NW = _NC * _NS


def _sc_mesh():
    return plsc.VectorSubcoreMesh(core_axis_name="c", subcore_axis_name="s",
                                  num_cores=_NC, num_subcores=_NS)


def _row_gather2(h, st, idx):
    bpw = TPAD // _NW
    ck = bpw // 2
    f32 = jnp.float32

    @functools.partial(
        pl.kernel, out_type=[jax.ShapeDtypeStruct((TPAD, C), f32)] * 2,
        mesh=_sc_mesh(),
        scratch_types=[pltpu.VMEM((bpw,), jnp.int32),
                       pltpu.VMEM((ck, C), f32),
                       pltpu.VMEM((ck, C), f32),
                       pltpu.SemaphoreType.DMA,
                       pltpu.SemaphoreType.DMA])
    def k(h_hbm, st_hbm, idx_hbm, oh_hbm, ost_hbm,
          idx_v, bh, bst, sem0, sem1):
        wid = lax.axis_index("s") * _NC + lax.axis_index("c")
        base = wid * bpw
        pltpu.sync_copy(idx_hbm.at[pl.ds(base, bpw)], idx_v)
        for c in range(2):
            off = base + c * ck
            idx_c = idx_v.at[pl.ds(c * ck, ck)]
            a0 = pltpu.async_copy(h_hbm.at[idx_c], bh, sem0)
            a1 = pltpu.async_copy(st_hbm.at[idx_c], bst, sem1)
            a0.wait()
            pltpu.sync_copy(bh, oh_hbm.at[pl.ds(off, ck)])
            a1.wait()
            pltpu.sync_copy(bst, ost_hbm.at[pl.ds(off, ck)])

    return k(h, st, idx)


def _combine_add(moe, x1, idx):
    bpw = T // _NW
    f32 = jnp.float32

    @functools.partial(
        pl.kernel, out_type=jax.ShapeDtypeStruct((T, C), f32),
        mesh=_sc_mesh(),
        scratch_types=[pltpu.VMEM((bpw,), jnp.int32),
                       pltpu.VMEM((bpw, C), f32),
                       pltpu.VMEM((bpw, C), f32),
                       pltpu.SemaphoreType.DMA])
    def k(moe_hbm, x1_hbm, idx_hbm, out_hbm, idx_v, mv, xv, sem):
        wid = lax.axis_index("s") * _NC + lax.axis_index("c")
        base = wid * bpw
        pltpu.sync_copy(idx_hbm.at[pl.ds(base, bpw)], idx_v)
        a = pltpu.async_copy(moe_hbm.at[idx_v], mv, sem)
        pltpu.sync_copy(x1_hbm.at[pl.ds(base, bpw)], xv)
        a.wait()

        def row(r, carry):
            for cc in range(C // 16):
                sl = pl.ds(cc * 16, 16)
                mv[r, sl] = mv[r, sl] + xv[r, sl]
            return carry

        lax.fori_loop(0, bpw, row, 0)
        pltpu.sync_copy(mv, out_hbm.at[pl.ds(base, bpw)])

    return k(moe, x1, idx)


def _row_gather1(src, idx, n_out):
    bpw = n_out // _NW
    f32 = jnp.float32

    @functools.partial(
        pl.kernel, out_type=jax.ShapeDtypeStruct((n_out, C), f32),
        mesh=_sc_mesh(),
        scratch_types=[pltpu.VMEM((bpw,), jnp.int32),
                       pltpu.VMEM((bpw, C), f32),
                       pltpu.SemaphoreType.DMA])
    def k(src_hbm, idx_hbm, out_hbm, idx_v, rows_v, sem):
        wid = lax.axis_index("s") * _NC + lax.axis_index("c")
        base = wid * bpw
        pltpu.sync_copy(idx_hbm.at[pl.ds(base, bpw)], idx_v)
        pltpu.async_copy(src_hbm.at[idx_v], rows_v, sem).wait()
        pltpu.sync_copy(rows_v, out_hbm.at[pl.ds(base, bpw)])

    return k(src, idx)


def _ln(z, g, b):
    m = jnp.mean(z, axis=-1, keepdims=True)
    v = jnp.mean((z - m) ** 2, axis=-1, keepdims=True)
    return (z - m) * lax.rsqrt(v + 1e-5) * g + b


def _prologue_body(x_ref, wr_ref, wk_ref, wv_ref, ws_ref, wo_ref,
                   wroute_ref, confb_ref,
                   cap_ref, ln1g_ref, ln1b_ref, ln2g_ref, ln2b_ref,
                   x1_ref, h_ref, st_ref, v_ref, win_ref, cost_ref, diff_ref,
                   aff_ref, scale_ref, rank_ref, cnt_ref, carry_ref):
    i = pl.program_id(0)
    g1, b1 = ln1g_ref[...], ln1b_ref[...]
    h1 = _ln(x_ref[...], g1, b1)
    h1s = jnp.concatenate([carry_ref[...], h1[:TR - 1, :]], axis=0)
    carry_ref[...] = h1[TR - 1:TR, :]
    row = lax.broadcasted_iota(jnp.int32, h1s.shape, 0) + i * TR
    h1s = jnp.where(row == 0, 0.0, h1s)
    mix = 0.5 * (h1 + h1s)
    r = jax.nn.sigmoid(jnp.dot(mix, wr_ref[...],
                               preferred_element_type=jnp.float32))
    k = jnp.dot(mix, wk_ref[...], preferred_element_type=jnp.float32)
    v = jnp.dot(mix, wv_ref[...], preferred_element_type=jnp.float32)
    st = jnp.dot(mix, ws_ref[...], preferred_element_type=jnp.float32)
    att = jnp.dot(r * k * v, wo_ref[...], preferred_element_type=jnp.float32)
    x1 = x_ref[...] + att
    h = _ln(x1, ln2g_ref[...], ln2b_ref[...])
    route = jnp.dot(h, wroute_ref[...], preferred_element_type=jnp.float32)
    conf = jax.nn.sigmoid(route[:, 0:E] + confb_ref[...])
    diff = jax.nn.sigmoid(route[:, E:E + 1])
    aff = route[:, E + 1:E + 1 + E]
    bids = conf * cap_ref[...] + 0.01 * aff
    maxb = jnp.max(bids, axis=-1, keepdims=True)
    eio = lax.broadcasted_iota(jnp.int32, bids.shape, 1)
    win = jnp.min(jnp.where(bids >= maxb, eio, E), axis=-1, keepdims=True)
    wb = jnp.sum(jnp.where(eio == win, conf, 0.0), axis=-1, keepdims=True)
    x1_ref[...] = x1
    h_ref[...] = h
    st_ref[...] = st
    v_ref[...] = v
    win_ref[...] = win
    cost_ref[...] = maxb * diff
    diff_ref[...] = diff
    aff_ref[...] = aff
    scale_ref[...] = wb / (wb + 1e-6)

    @pl.when(i == 0)
    def _init():
        cnt_ref[...] = jnp.zeros_like(cnt_ref)

    oh = (eio == win).astype(jnp.float32)
    rio = lax.broadcasted_iota(jnp.int32, (TR, TR), 0)
    cio = lax.broadcasted_iota(jnp.int32, (TR, TR), 1)
    tri = (rio > cio).astype(jnp.float32)
    excl = jnp.dot(tri, oh, preferred_element_type=jnp.float32)
    base = cnt_ref[...].astype(jnp.float32)
    rank_ref[...] = jnp.sum(oh * (excl + base), axis=1,
                            keepdims=True).astype(jnp.int32)
    cnt_ref[...] += jnp.sum(oh, axis=0, keepdims=True).astype(jnp.int32)


def _prologue(x2d, wr, wk, wv, ws, wo, wroute, confb, cap, g1, b1, g2, b2):
    rows = lambda i: (i, 0)
    whole = lambda i: (0, 0)
    f32 = jnp.float32
    return pl.pallas_call(
        _prologue_body,
        grid=(T // TR,),
        in_specs=[
            pl.BlockSpec((TR, C), rows),
            pl.BlockSpec((C, C), whole),
            pl.BlockSpec((C, C), whole),
            pl.BlockSpec((C, C), whole),
            pl.BlockSpec((C, C), whole),
            pl.BlockSpec((C, C), whole),
            pl.BlockSpec((C, 2 * E + 1), whole),
            pl.BlockSpec((1, E), whole),
            pl.BlockSpec((1, E), whole),
            pl.BlockSpec((1, C), whole),
            pl.BlockSpec((1, C), whole),
            pl.BlockSpec((1, C), whole),
            pl.BlockSpec((1, C), whole),
        ],
        out_specs=[
            pl.BlockSpec((TR, C), rows),
            pl.BlockSpec((TR, C), rows),
            pl.BlockSpec((TR, C), rows),
            pl.BlockSpec((TR, C), rows),
            pl.BlockSpec((TR, 1), rows),
            pl.BlockSpec((TR, 1), rows),
            pl.BlockSpec((TR, 1), rows),
            pl.BlockSpec((TR, E), rows),
            pl.BlockSpec((TR, 1), rows),
            pl.BlockSpec((TR, 1), rows),
            pl.BlockSpec((1, E), whole),
        ],
        out_shape=[
            jax.ShapeDtypeStruct((T, C), f32),
            jax.ShapeDtypeStruct((T, C), f32),
            jax.ShapeDtypeStruct((T, C), f32),
            jax.ShapeDtypeStruct((T, C), f32),
            jax.ShapeDtypeStruct((T, 1), jnp.int32),
            jax.ShapeDtypeStruct((T, 1), f32),
            jax.ShapeDtypeStruct((T, 1), f32),
            jax.ShapeDtypeStruct((T, E), f32),
            jax.ShapeDtypeStruct((T, 1), f32),
            jax.ShapeDtypeStruct((T, 1), jnp.int32),
            jax.ShapeDtypeStruct((1, E), jnp.int32),
        ],
        scratch_shapes=[pltpu.VMEM((1, C), f32)],
    )(x2d, wr, wk, wv, ws, wo, wroute, confb, cap, g1, b1, g2, b2)


def _ffn(tile_expert, row_limit, h_s, st_s, sc_s, w1, b1e, w2, b2e, ws1, wrec):
    f32 = jnp.float32
    look = pl.Buffered(buffer_count=2, use_lookahead=True)

    def inner(te_ref, rl_ref, h_hbm, st_hbm, sc_hbm, w1_hbm, b1_hbm,
              w2_hbm, b2_hbm, ws1_ref, wrec_ref, out_hbm, rec_hbm):
        rows = lambda i: (i, 0)
        byexp3 = lambda i: (te_ref[i], 0, 0)
        strows = lambda i: (jnp.where(te_ref[i] == E - 1, i, 0), 0)

        def kbody(idx, h_ref, st_ref, sc_ref, w1_ref, b1_ref,
                  w2_ref, b2_ref, out_ref, rec_ref):
            i = idx[0]
            e = te_ref[i]
            h = h_ref[...]
            st = st_ref[...]
            sel = (e == E - 1).astype(f32)
            base = (jnp.dot(h, w1_ref[0], preferred_element_type=f32)
                    + sel * jnp.dot(st, ws1_ref[...],
                                    preferred_element_type=f32)
                    + b1_ref[0])
            hr = jax.nn.relu(base)
            hid = jnp.where(e == E - 1, hr, hr * hr)
            out = jnp.dot(hid, w2_ref[0], preferred_element_type=f32) + b2_ref[0]
            out_ref[...] = out * sc_ref[...]
            rr = jnp.dot(h, wrec_ref[...], preferred_element_type=f32) - st
            rowid = lax.broadcasted_iota(jnp.int32, (TM, 1), 0) + i * TM
            vm = (rowid < rl_ref[i]).astype(f32)
            part = jnp.sum(jnp.sum(rr * rr, axis=-1, keepdims=True) * vm)
            rec_ref[...] = jnp.where(i == 0, 0.0,
                                     rec_ref[...]) + part.reshape(1, 1)

        pipeline = pltpu.emit_pipeline(
            kbody,
            grid=(NT,),
            in_specs=[
                pl.BlockSpec((TM, C), rows),
                pl.BlockSpec((TM, C), strows),
                pl.BlockSpec((TM, 1), rows),
                pl.BlockSpec((1, C, H), byexp3, pipeline_mode=look),
                pl.BlockSpec((1, 1, H), byexp3, pipeline_mode=look),
                pl.BlockSpec((1, H, C), byexp3, pipeline_mode=look),
                pl.BlockSpec((1, 1, C), byexp3, pipeline_mode=look),
            ],
            out_specs=[pl.BlockSpec((TM, C), rows),
                       pl.BlockSpec((1, 1), lambda i: (0, 0))],
            _explicit_indices=True,
        )
        pipeline(h_hbm, st_hbm, sc_hbm, w1_hbm, b1_hbm, w2_hbm,
                 b2_hbm, out_hbm, rec_hbm)

    anyspace = pl.BlockSpec(memory_space=pl.ANY)
    return pl.pallas_call(
        inner,
        in_specs=[
            pl.BlockSpec(memory_space=pltpu.SMEM),
            pl.BlockSpec(memory_space=pltpu.SMEM),
            anyspace, anyspace, anyspace, anyspace, anyspace, anyspace,
            anyspace,
            pl.BlockSpec(memory_space=pltpu.VMEM),
            pl.BlockSpec(memory_space=pltpu.VMEM),
        ],
        out_specs=[anyspace, anyspace],
        out_shape=[jax.ShapeDtypeStruct((TPAD, C), f32),
                   jax.ShapeDtypeStruct((1, 1), f32)],
    )(tile_expert, row_limit, h_s, st_s, sc_s, w1, b1e, w2, b2e, ws1, wrec)


def kernel(x, v_first, capital_shares, ln1_g, ln1_b, ln2_g, ln2_b, Wr, Wk, Wv,
           Wo, Ws, W1, b1, W2, b2, Ws1, Wrec, conf_w, conf_b, Wd, Wa):
    f32 = jnp.float32
    x2d = x.reshape(T, C)
    wroute = jnp.concatenate([conf_w.T, Wd, Wa], axis=1)

    (x1, h, st, v, win2, cost2, diff2, aff, scale2, rank2, cnt2) = \
        _prologue(
            x2d, Wr, Wk, Wv, Ws, Wo, wroute, conf_b.reshape(1, E),
            capital_shares.reshape(1, E), ln1_g.reshape(1, C),
            ln1_b.reshape(1, C), ln2_g.reshape(1, C), ln2_b.reshape(1, C))

    winners = win2[:, 0]
    counts = cnt2[0]
    tiles_e = (counts + TM - 1) // TM
    cum_tiles = jnp.cumsum(tiles_e)
    pstart = (cum_tiles - tiles_e) * TM
    ti = jnp.arange(NT)
    tile_expert = jnp.minimum(
        jnp.sum((ti[:, None] >= cum_tiles[None, :]).astype(jnp.int32), axis=1),
        E - 1).astype(jnp.int32)
    inv_perm = (pstart[winners] + rank2[:, 0]).astype(jnp.int32)
    qi = jnp.arange(TPAD)
    packed = jnp.zeros((TPAD,), jnp.int32).at[inv_perm].set(
        jnp.arange(T, dtype=jnp.int32) + 1)
    src_row = jnp.where(packed > 0, packed - 1, qi % T).astype(jnp.int32)
    sc_s = scale2[src_row]

    h_s, st_s = _row_gather2(h, st, src_row)

    row_limit = jnp.where(tile_expert == E - 1,
                          pstart[E - 1] + counts[E - 1], 0).astype(jnp.int32)
    ffn_out, rec_sum = _ffn(tile_expert, row_limit, h_s, st_s, sc_s,
                            W1, b1.reshape(E, 1, H), W2, b2.reshape(E, 1, C),
                            Ws1, Wrec)

    x_out = _combine_add(ffn_out, x1, inv_perm)

    cnt7 = counts[E - 1]
    recon = jnp.where(cnt7 > 0, rec_sum[0, 0] / (cnt7 * C).astype(f32), 0.0)

    return (x_out.reshape(1, T, C), v.reshape(1, T, C), winners.reshape(1, T),
            cost2[:, 0].reshape(1, T), diff2.reshape(1, T, 1),
            aff.reshape(1, T, E), recon)


# speedup vs baseline: 2.2390x; 1.0115x over previous
import functools

import jax
import jax.numpy as jnp
from jax import lax
from jax.experimental import pallas as pl
from jax.experimental.pallas import tpu as pltpu
from jax.experimental.pallas import tpu_sc as plsc

T = 2048
C = 768
E = 8
H = 4 * C
TM = 128
NT = T // TM + 8
TPAD = NT * TM
TR = 256


_NC, _NS = 2, 16
_NW = _NC * _NS


def _sc_mesh():
    return plsc.VectorSubcoreMesh(core_axis_name="c", subcore_axis_name="s",
                                  num_cores=_NC, num_subcores=_NS)


def _row_gather2(h, st, idx):
    bpw = TPAD // _NW
    ck = bpw // 2
    f32 = jnp.float32

    @functools.partial(
        pl.kernel, out_type=[jax.ShapeDtypeStruct((TPAD, C), f32)] * 2,
        mesh=_sc_mesh(),
        scratch_types=[pltpu.VMEM((bpw,), jnp.int32),
                       pltpu.VMEM((ck, C), f32),
                       pltpu.VMEM((ck, C), f32),
                       pltpu.SemaphoreType.DMA,
                       pltpu.SemaphoreType.DMA])
    def k(h_hbm, st_hbm, idx_hbm, oh_hbm, ost_hbm,
          idx_v, bh, bst, sem0, sem1):
        wid = lax.axis_index("s") * _NC + lax.axis_index("c")
        base = wid * bpw
        pltpu.sync_copy(idx_hbm.at[pl.ds(base, bpw)], idx_v)
        for c in range(2):
            off = base + c * ck
            idx_c = idx_v.at[pl.ds(c * ck, ck)]
            a0 = pltpu.async_copy(h_hbm.at[idx_c], bh, sem0)
            a1 = pltpu.async_copy(st_hbm.at[idx_c], bst, sem1)
            a0.wait()
            pltpu.sync_copy(bh, oh_hbm.at[pl.ds(off, ck)])
            a1.wait()
            pltpu.sync_copy(bst, ost_hbm.at[pl.ds(off, ck)])

    return k(h, st, idx)


def _combine_add(moe, x1, idx):
    bpw = T // _NW
    f32 = jnp.float32

    @functools.partial(
        pl.kernel, out_type=jax.ShapeDtypeStruct((T, C), f32),
        mesh=_sc_mesh(),
        scratch_types=[pltpu.VMEM((bpw,), jnp.int32),
                       pltpu.VMEM((bpw, C), f32),
                       pltpu.VMEM((bpw, C), f32),
                       pltpu.SemaphoreType.DMA])
    def k(moe_hbm, x1_hbm, idx_hbm, out_hbm, idx_v, mv, xv, sem):
        wid = lax.axis_index("s") * _NC + lax.axis_index("c")
        base = wid * bpw
        pltpu.sync_copy(idx_hbm.at[pl.ds(base, bpw)], idx_v)
        a = pltpu.async_copy(moe_hbm.at[idx_v], mv, sem)
        pltpu.sync_copy(x1_hbm.at[pl.ds(base, bpw)], xv)
        a.wait()

        def row(r, carry):
            for cc in range(C // 16):
                sl = pl.ds(cc * 16, 16)
                mv[r, sl] = mv[r, sl] + xv[r, sl]
            return carry

        lax.fori_loop(0, bpw, row, 0)
        pltpu.sync_copy(mv, out_hbm.at[pl.ds(base, bpw)])

    return k(moe, x1, idx)


def _row_gather1(src, idx, n_out):
    bpw = n_out // _NW
    f32 = jnp.float32

    @functools.partial(
        pl.kernel, out_type=jax.ShapeDtypeStruct((n_out, C), f32),
        mesh=_sc_mesh(),
        scratch_types=[pltpu.VMEM((bpw,), jnp.int32),
                       pltpu.VMEM((bpw, C), f32),
                       pltpu.SemaphoreType.DMA])
    def k(src_hbm, idx_hbm, out_hbm, idx_v, rows_v, sem):
        wid = lax.axis_index("s") * _NC + lax.axis_index("c")
        base = wid * bpw
        pltpu.sync_copy(idx_hbm.at[pl.ds(base, bpw)], idx_v)
        pltpu.async_copy(src_hbm.at[idx_v], rows_v, sem).wait()
        pltpu.sync_copy(rows_v, out_hbm.at[pl.ds(base, bpw)])

    return k(src, idx)


def _ln(z, g, b):
    m = jnp.mean(z, axis=-1, keepdims=True)
    v = jnp.mean((z - m) ** 2, axis=-1, keepdims=True)
    return (z - m) * lax.rsqrt(v + 1e-5) * g + b


def _prologue_body(x_ref, wr_ref, wk_ref, wv_ref, ws_ref, wo_ref,
                   wroute_ref, confb_ref,
                   cap_ref, ln1g_ref, ln1b_ref, ln2g_ref, ln2b_ref,
                   x1_ref, h_ref, st_ref, v_ref, win_ref, cost_ref, diff_ref,
                   aff_ref, scale_ref, rank_ref, cnt_ref, carry_ref):
    i = pl.program_id(0)
    g1, b1 = ln1g_ref[...], ln1b_ref[...]
    h1 = _ln(x_ref[...], g1, b1)
    h1s = jnp.concatenate([carry_ref[...], h1[:TR - 1, :]], axis=0)
    carry_ref[...] = h1[TR - 1:TR, :]
    row = lax.broadcasted_iota(jnp.int32, h1s.shape, 0) + i * TR
    h1s = jnp.where(row == 0, 0.0, h1s)
    mix = 0.5 * (h1 + h1s)
    r = jax.nn.sigmoid(jnp.dot(mix, wr_ref[...],
                               preferred_element_type=jnp.float32))
    k = jnp.dot(mix, wk_ref[...], preferred_element_type=jnp.float32)
    v = jnp.dot(mix, wv_ref[...], preferred_element_type=jnp.float32)
    st = jnp.dot(mix, ws_ref[...], preferred_element_type=jnp.float32)
    att = jnp.dot(r * k * v, wo_ref[...], preferred_element_type=jnp.float32)
    x1 = x_ref[...] + att
    h = _ln(x1, ln2g_ref[...], ln2b_ref[...])
    route = jnp.dot(h, wroute_ref[...], preferred_element_type=jnp.float32)
    conf = jax.nn.sigmoid(route[:, 0:E] + confb_ref[...])
    diff = jax.nn.sigmoid(route[:, E:E + 1])
    aff = route[:, E + 1:E + 1 + E]
    bids = conf * cap_ref[...] + 0.01 * aff
    maxb = jnp.max(bids, axis=-1, keepdims=True)
    eio = lax.broadcasted_iota(jnp.int32, bids.shape, 1)
    win = jnp.min(jnp.where(bids >= maxb, eio, E), axis=-1, keepdims=True)
    wb = jnp.sum(jnp.where(eio == win, conf, 0.0), axis=-1, keepdims=True)
    x1_ref[...] = x1
    h_ref[...] = h
    st_ref[...] = st
    v_ref[...] = v
    win_ref[...] = win
    cost_ref[...] = maxb * diff
    diff_ref[...] = diff
    aff_ref[...] = aff
    scale_ref[...] = wb / (wb + 1e-6)

    @pl.when(i == 0)
    def _init():
        cnt_ref[...] = jnp.zeros_like(cnt_ref)

    oh = (eio == win).astype(jnp.float32)
    rio = lax.broadcasted_iota(jnp.int32, (TR, TR), 0)
    cio = lax.broadcasted_iota(jnp.int32, (TR, TR), 1)
    tri = (rio > cio).astype(jnp.float32)
    excl = jnp.dot(tri, oh, preferred_element_type=jnp.float32)
    base = cnt_ref[...].astype(jnp.float32)
    rank_ref[...] = jnp.sum(oh * (excl + base), axis=1,
                            keepdims=True).astype(jnp.int32)
    cnt_ref[...] += jnp.sum(oh, axis=0, keepdims=True).astype(jnp.int32)


def _prologue(x2d, wr, wk, wv, ws, wo, wroute, confb, cap, g1, b1, g2, b2):
    rows = lambda i: (i, 0)
    whole = lambda i: (0, 0)
    f32 = jnp.float32
    return pl.pallas_call(
        _prologue_body,
        grid=(T // TR,),
        in_specs=[
            pl.BlockSpec((TR, C), rows),
            pl.BlockSpec((C, C), whole),
            pl.BlockSpec((C, C), whole),
            pl.BlockSpec((C, C), whole),
            pl.BlockSpec((C, C), whole),
            pl.BlockSpec((C, C), whole),
            pl.BlockSpec((C, 2 * E + 1), whole),
            pl.BlockSpec((1, E), whole),
            pl.BlockSpec((1, E), whole),
            pl.BlockSpec((1, C), whole),
            pl.BlockSpec((1, C), whole),
            pl.BlockSpec((1, C), whole),
            pl.BlockSpec((1, C), whole),
        ],
        out_specs=[
            pl.BlockSpec((TR, C), rows),
            pl.BlockSpec((TR, C), rows),
            pl.BlockSpec((TR, C), rows),
            pl.BlockSpec((TR, C), rows),
            pl.BlockSpec((TR, 1), rows),
            pl.BlockSpec((TR, 1), rows),
            pl.BlockSpec((TR, 1), rows),
            pl.BlockSpec((TR, E), rows),
            pl.BlockSpec((TR, 1), rows),
            pl.BlockSpec((TR, 1), rows),
            pl.BlockSpec((1, E), whole),
        ],
        out_shape=[
            jax.ShapeDtypeStruct((T, C), f32),
            jax.ShapeDtypeStruct((T, C), f32),
            jax.ShapeDtypeStruct((T, C), f32),
            jax.ShapeDtypeStruct((T, C), f32),
            jax.ShapeDtypeStruct((T, 1), jnp.int32),
            jax.ShapeDtypeStruct((T, 1), f32),
            jax.ShapeDtypeStruct((T, 1), f32),
            jax.ShapeDtypeStruct((T, E), f32),
            jax.ShapeDtypeStruct((T, 1), f32),
            jax.ShapeDtypeStruct((T, 1), jnp.int32),
            jax.ShapeDtypeStruct((1, E), jnp.int32),
        ],
        scratch_shapes=[pltpu.VMEM((1, C), f32)],
    )(x2d, wr, wk, wv, ws, wo, wroute, confb, cap, g1, b1, g2, b2)


def _ffn(tile_expert, row_limit, h_s, st_s, sc_s, w1, b1e, w2, b2e, ws1, wrec):
    f32 = jnp.float32
    look = pl.Buffered(buffer_count=2, use_lookahead=True)

    def inner(te_ref, rl_ref, h_hbm, st_hbm, sc_hbm, w1_hbm, b1_hbm,
              w2_hbm, b2_hbm, ws1_ref, wrec_ref, out_hbm, rec_hbm):
        rows = lambda i: (i, 0)
        byexp3 = lambda i: (te_ref[i], 0, 0)
        strows = lambda i: (jnp.where(te_ref[i] == E - 1, i, 0), 0)

        def kbody(idx, h_ref, st_ref, sc_ref, w1_ref, b1_ref,
                  w2_ref, b2_ref, out_ref, rec_ref):
            i = idx[0]
            e = te_ref[i]
            h = h_ref[...]
            st = st_ref[...]
            sel = (e == E - 1).astype(f32)
            base = (jnp.dot(h, w1_ref[0], preferred_element_type=f32)
                    + sel * jnp.dot(st, ws1_ref[...],
                                    preferred_element_type=f32)
                    + b1_ref[0])
            hr = jax.nn.relu(base)
            hid = jnp.where(e == E - 1, hr, hr * hr)
            out = jnp.dot(hid, w2_ref[0], preferred_element_type=f32) + b2_ref[0]
            out_ref[...] = out * sc_ref[...]
            rr = jnp.dot(h, wrec_ref[...], preferred_element_type=f32) - st
            rowid = lax.broadcasted_iota(jnp.int32, (TM, 1), 0) + i * TM
            vm = (rowid < rl_ref[i]).astype(f32)
            part = jnp.sum(jnp.sum(rr * rr, axis=-1, keepdims=True) * vm)
            rec_ref[...] = jnp.where(i == 0, 0.0,
                                     rec_ref[...]) + part.reshape(1, 1)

        pipeline = pltpu.emit_pipeline(
            kbody,
            grid=(NT,),
            in_specs=[
                pl.BlockSpec((TM, C), rows),
                pl.BlockSpec((TM, C), strows),
                pl.BlockSpec((TM, 1), rows),
                pl.BlockSpec((1, C, H), byexp3, pipeline_mode=look),
                pl.BlockSpec((1, 1, H), byexp3, pipeline_mode=look),
                pl.BlockSpec((1, H, C), byexp3, pipeline_mode=look),
                pl.BlockSpec((1, 1, C), byexp3, pipeline_mode=look),
            ],
            out_specs=[pl.BlockSpec((TM, C), rows),
                       pl.BlockSpec((1, 1), lambda i: (0, 0))],
            _explicit_indices=True,
        )
        pipeline(h_hbm, st_hbm, sc_hbm, w1_hbm, b1_hbm, w2_hbm,
                 b2_hbm, out_hbm, rec_hbm)

    anyspace = pl.BlockSpec(memory_space=pl.ANY)
    return pl.pallas_call(
        inner,
        in_specs=[
            pl.BlockSpec(memory_space=pltpu.SMEM),
            pl.BlockSpec(memory_space=pltpu.SMEM),
            anyspace, anyspace, anyspace, anyspace, anyspace, anyspace,
            anyspace,
            pl.BlockSpec(memory_space=pltpu.VMEM),
            pl.BlockSpec(memory_space=pltpu.VMEM),
        ],
        out_specs=[anyspace, anyspace],
        out_shape=[jax.ShapeDtypeStruct((TPAD, C), f32),
                   jax.ShapeDtypeStruct((1, 1), f32)],
    )(tile_expert, row_limit, h_s, st_s, sc_s, w1, b1e, w2, b2e, ws1, wrec)


def kernel(x, v_first, capital_shares, ln1_g, ln1_b, ln2_g, ln2_b, Wr, Wk, Wv,
           Wo, Ws, W1, b1, W2, b2, Ws1, Wrec, conf_w, conf_b, Wd, Wa):
    f32 = jnp.float32
    x2d = x.reshape(T, C)
    wroute = jnp.concatenate([conf_w.T, Wd, Wa], axis=1)

    (x1, h, st, v, win2, cost2, diff2, aff, scale2, rank2, cnt2) = \
        _prologue(
            x2d, Wr, Wk, Wv, Ws, Wo, wroute, conf_b.reshape(1, E),
            capital_shares.reshape(1, E), ln1_g.reshape(1, C),
            ln1_b.reshape(1, C), ln2_g.reshape(1, C), ln2_b.reshape(1, C))

    winners = win2[:, 0]
    counts = cnt2[0]
    tiles_e = (counts + TM - 1) // TM
    cum_tiles = jnp.cumsum(tiles_e)
    pstart = (cum_tiles - tiles_e) * TM
    ti = jnp.arange(NT)
    tile_expert = jnp.minimum(
        jnp.sum((ti[:, None] >= cum_tiles[None, :]).astype(jnp.int32), axis=1),
        E - 1).astype(jnp.int32)
    inv_perm = (pstart[winners] + rank2[:, 0]).astype(jnp.int32)
    src_row = (jnp.arange(TPAD, dtype=jnp.int32) % T).at[inv_perm].set(
        jnp.arange(T, dtype=jnp.int32))
    sc_s = scale2[src_row]

    h_s, st_s = _row_gather2(h, st, src_row)

    row_limit = jnp.where(tile_expert == E - 1,
                          pstart[E - 1] + counts[E - 1], 0).astype(jnp.int32)
    ffn_out, rec_sum = _ffn(tile_expert, row_limit, h_s, st_s, sc_s,
                            W1, b1.reshape(E, 1, H), W2, b2.reshape(E, 1, C),
                            Ws1, Wrec)

    x_out = _combine_add(ffn_out, x1, inv_perm)

    cnt7 = counts[E - 1]
    recon = jnp.where(cnt7 > 0, rec_sum[0, 0] / (cnt7 * C).astype(f32), 0.0)

    return (x_out.reshape(1, T, C), v.reshape(1, T, C), winners.reshape(1, T),
            cost2[:, 0].reshape(1, T), diff2.reshape(1, T, 1),
            aff.reshape(1, T, E), recon)
